# Initial kernel scaffold; baseline (speedup 1.0000x reference)
import jax
import jax.numpy as jnp
from jax.experimental import pallas as pl


def kernel(x, edge_index, edge_attr, u, batch, agg_w1, agg_b1, agg_w2, agg_b2, upd_w1, upd_b1, upd_w2, upd_b2, gn_alpha, gn_gamma, gn_beta):
    raise NotImplementedError("write your pallas kernel here")


# speedup vs baseline: 2.6206x; 2.6206x over previous
import functools

import jax
import jax.numpy as jnp
from jax import lax
from jax.experimental import pallas as pl
from jax.experimental.pallas import tpu as pltpu
from jax.experimental.pallas import tpu_sc as plsc

_NC = 2
_NS = 16
_NW = _NC * _NS
_K = 128
_ZCH = 5
_NT = _K * _ZCH
_NPAD = _NT * _NS


def _sc_mesh():
    return plsc.VectorSubcoreMesh(
        core_axis_name="c", subcore_axis_name="s",
        num_cores=_NC, num_subcores=_NS)


def _sc_gather_count(x, src, dst, n, e, d):
    nch = e // _K
    consts = jnp.stack([jnp.zeros((_K, d), jnp.float32),
                        jnp.ones((_K, d), jnp.float32)])

    @functools.partial(
        pl.kernel,
        out_type=(jax.ShapeDtypeStruct((nch, _K, d), jnp.float32),
                  jax.ShapeDtypeStruct((_NC * _NPAD, d), jnp.float32)),
        mesh=_sc_mesh(),
        scratch_types=[
            pltpu.VMEM((_K,), jnp.int32),
            pltpu.VMEM((_K,), jnp.int32),
            pltpu.VMEM((_K, d), jnp.float32),
            pltpu.VMEM((_K, d), jnp.float32),
            pltpu.VMEM_SHARED((_NPAD, d), jnp.float32),
            pltpu.SemaphoreType.DMA,
        ],
    )
    def gk(x_hbm, src_hbm, dst_hbm, const_hbm, out_hbm, cnt_hbm,
           idx_v, idx2_v, rows_v, ones_v, cnt_sh, sem):
        cid = lax.axis_index("c")
        sid = lax.axis_index("s")
        wid = cid * _NS + sid
        nj = nch // _NW + jnp.where(wid < nch % _NW, 1, 0)

        pltpu.sync_copy(const_hbm.at[0], rows_v)
        pltpu.sync_copy(const_hbm.at[1], ones_v)

        def zbody(z, carry):
            pltpu.sync_copy(rows_v, cnt_sh.at[pl.ds(sid * _NT + z * _K, _K)])
            return carry

        lax.fori_loop(0, _ZCH, zbody, 0)
        plsc.subcore_barrier()

        def body(j, carry):
            c = wid + j * _NW
            pltpu.sync_copy(src_hbm.at[pl.ds(c * _K, _K)], idx_v)
            pltpu.async_copy(x_hbm.at[idx_v], rows_v, sem).wait()
            pltpu.sync_copy(rows_v, out_hbm.at[c])
            pltpu.sync_copy(dst_hbm.at[pl.ds(c * _K, _K)], idx2_v)
            pltpu.sync_copy(ones_v, cnt_sh.at[idx2_v], add=True)
            return carry

        lax.fori_loop(0, nj, body, 0)
        plsc.subcore_barrier()

        def wbody(z, carry):
            off = sid * _NT + z * _K
            pltpu.sync_copy(cnt_sh.at[pl.ds(off, _K)], rows_v)
            pltpu.sync_copy(rows_v, cnt_hbm.at[pl.ds(cid * _NPAD + off, _K)])
            return carry

        lax.fori_loop(0, _ZCH, wbody, 0)

    xs, cnt = gk(x, src, dst, consts)
    return xs.reshape(e, d), cnt.reshape(_NC, _NPAD, d)[:, :n]


def _sc_scatter(msg, dst, n, e, d):
    nch = e // _K
    msg3 = msg.reshape(nch, _K, d)
    zeros = jnp.zeros((_K, d), jnp.float32)

    @functools.partial(
        pl.kernel,
        out_type=jax.ShapeDtypeStruct((_NC * _NPAD, d), jnp.float32),
        mesh=_sc_mesh(),
        scratch_types=[
            pltpu.VMEM((_K,), jnp.int32),
            pltpu.VMEM((_K, d), jnp.float32),
            pltpu.VMEM_SHARED((_NPAD, d), jnp.float32),
            pltpu.SemaphoreType.DMA,
        ],
    )
    def sk(msg_hbm, dst_hbm, zero_hbm, out_hbm, idx_v, buf_v, acc_sh, sem):
        cid = lax.axis_index("c")
        sid = lax.axis_index("s")
        wid = cid * _NS + sid
        nj = nch // _NW + jnp.where(wid < nch % _NW, 1, 0)

        pltpu.sync_copy(zero_hbm, buf_v)

        def zbody(z, carry):
            pltpu.sync_copy(buf_v, acc_sh.at[pl.ds(sid * _NT + z * _K, _K)])
            return carry

        lax.fori_loop(0, _ZCH, zbody, 0)
        plsc.subcore_barrier()

        def body(j, carry):
            c = wid + j * _NW
            pltpu.sync_copy(dst_hbm.at[pl.ds(c * _K, _K)], idx_v)
            pltpu.async_copy(msg_hbm.at[c], buf_v, sem).wait()
            pltpu.sync_copy(buf_v, acc_sh.at[idx_v], add=True)
            return carry

        lax.fori_loop(0, nj, body, 0)
        plsc.subcore_barrier()

        def wbody(z, carry):
            off = sid * _NT + z * _K
            pltpu.sync_copy(acc_sh.at[pl.ds(off, _K)], buf_v)
            pltpu.sync_copy(buf_v, out_hbm.at[pl.ds(cid * _NPAD + off, _K)])
            return carry

        lax.fori_loop(0, _ZCH, wbody, 0)

    return sk(msg3, dst, zeros).reshape(_NC, _NPAD, d)[:, :n]


def _edge_mlp(xs, ea, w1a, w1b, b1, w2, b2, e, d, h):
    be = 2000
    grid = e // be

    def body(xs_ref, ea_ref, w1a_ref, w1b_ref, b1_ref, w2_ref, b2_ref, o_ref):
        hv = (jnp.dot(xs_ref[...], w1a_ref[...], preferred_element_type=jnp.float32)
              + jnp.dot(ea_ref[...], w1b_ref[...], preferred_element_type=jnp.float32)
              + b1_ref[...])
        hv = jnp.maximum(hv, 0.0)
        o_ref[...] = (jnp.dot(hv, w2_ref[...], preferred_element_type=jnp.float32)
                      + b2_ref[...])

    return pl.pallas_call(
        body,
        grid=(grid,),
        in_specs=[
            pl.BlockSpec((be, d), lambda i: (i, 0)),
            pl.BlockSpec((be, d), lambda i: (i, 0)),
            pl.BlockSpec((d, h), lambda i: (0, 0)),
            pl.BlockSpec((d, h), lambda i: (0, 0)),
            pl.BlockSpec((1, h), lambda i: (0, 0)),
            pl.BlockSpec((h, d), lambda i: (0, 0)),
            pl.BlockSpec((1, d), lambda i: (0, 0)),
        ],
        out_specs=pl.BlockSpec((be, d), lambda i: (i, 0)),
        out_shape=jax.ShapeDtypeStruct((e, d), jnp.float32),
    )(xs, ea, w1a, w1b, b1, w2, b2)


def _node_mlp(x, p0, p1, c0, c1, u, batch3, w1a, w1b, w1c, b1, w2, b2,
              n, d, nb, gd, h):
    bn = 1000
    grid = n // bn

    def body(x_ref, p0_ref, p1_ref, c0_ref, c1_ref, u_ref, b_ref,
             w1a_ref, w1b_ref, w1c_ref, b1_ref, w2_ref, b2_ref,
             y_ref, s0_ref, s1_ref, s2_ref):
        i = pl.program_id(0)
        xv = x_ref[...]
        p = p0_ref[...] + p1_ref[...]
        cnt = jnp.max(c0_ref[...] + c1_ref[...], axis=1, keepdims=True)
        agg = p / jnp.clip(cnt, 1.0, None)
        bv = b_ref[0, 0, :]
        oh = (bv[:, None] == lax.broadcasted_iota(jnp.int32, (bn, nb), 1)
              ).astype(jnp.float32)
        ub = jnp.dot(oh, u_ref[...], preferred_element_type=jnp.float32)
        hv = (jnp.dot(xv, w1a_ref[...], preferred_element_type=jnp.float32)
              + jnp.dot(agg, w1b_ref[...], preferred_element_type=jnp.float32)
              + jnp.dot(ub, w1c_ref[...], preferred_element_type=jnp.float32)
              + b1_ref[...])
        hv = jnp.maximum(hv, 0.0)
        y = (jnp.dot(hv, w2_ref[...], preferred_element_type=jnp.float32)
             + b2_ref[...] + xv)
        y_ref[...] = y

        dg = lambda a, b: lax.dot_general(
            a, b, (((0,), (0,)), ((), ())), preferred_element_type=jnp.float32)

        @pl.when(i == 0)
        def _():
            s0_ref[...] = jnp.zeros_like(s0_ref)
            s1_ref[...] = jnp.zeros_like(s1_ref)
            s2_ref[...] = jnp.zeros_like(s2_ref)

        s0_ref[...] += dg(oh, jnp.ones_like(y))
        s1_ref[...] += dg(oh, y)
        s2_ref[...] += dg(oh, y * y)

    return pl.pallas_call(
        body,
        grid=(grid,),
        in_specs=[
            pl.BlockSpec((bn, d), lambda i: (i, 0)),
            pl.BlockSpec((bn, d), lambda i: (i, 0)),
            pl.BlockSpec((bn, d), lambda i: (i, 0)),
            pl.BlockSpec((bn, d), lambda i: (i, 0)),
            pl.BlockSpec((bn, d), lambda i: (i, 0)),
            pl.BlockSpec((nb, gd), lambda i: (0, 0)),
            pl.BlockSpec((1, 1, bn), lambda i: (i, 0, 0)),
            pl.BlockSpec((d, h), lambda i: (0, 0)),
            pl.BlockSpec((d, h), lambda i: (0, 0)),
            pl.BlockSpec((gd, h), lambda i: (0, 0)),
            pl.BlockSpec((1, h), lambda i: (0, 0)),
            pl.BlockSpec((h, d), lambda i: (0, 0)),
            pl.BlockSpec((1, d), lambda i: (0, 0)),
        ],
        out_specs=[
            pl.BlockSpec((bn, d), lambda i: (i, 0)),
            pl.BlockSpec((nb, d), lambda i: (0, 0)),
            pl.BlockSpec((nb, d), lambda i: (0, 0)),
            pl.BlockSpec((nb, d), lambda i: (0, 0)),
        ],
        out_shape=[
            jax.ShapeDtypeStruct((n, d), jnp.float32),
            jax.ShapeDtypeStruct((nb, d), jnp.float32),
            jax.ShapeDtypeStruct((nb, d), jnp.float32),
            jax.ShapeDtypeStruct((nb, d), jnp.float32),
        ],
    )(x, p0, p1, c0, c1, u, batch3, w1a, w1b, w1c, b1, w2, b2)


def _graphnorm(y, batch3, s0, s1, s2, alpha, gamma, beta, n, d, nb):
    bn = 1000
    grid = n // bn

    def body(y_ref, b_ref, s0_ref, s1_ref, s2_ref, al_ref, ga_ref, be_ref,
             o_ref):
        gcnt = jnp.clip(s0_ref[...], 1.0, None)
        mean = s1_ref[...] / gcnt
        am = al_ref[...] * mean
        var = s2_ref[...] / gcnt - am * (2.0 * mean - am)
        scale = ga_ref[...] / jnp.sqrt(var + 1e-5)
        shift = be_ref[...] - scale * am
        bv = b_ref[0, 0, :]
        oh = (bv[:, None] == lax.broadcasted_iota(jnp.int32, (bn, nb), 1)
              ).astype(jnp.float32)
        o_ref[...] = (jnp.dot(oh, scale, preferred_element_type=jnp.float32)
                      * y_ref[...]
                      + jnp.dot(oh, shift, preferred_element_type=jnp.float32))

    return pl.pallas_call(
        body,
        grid=(grid,),
        in_specs=[
            pl.BlockSpec((bn, d), lambda i: (i, 0)),
            pl.BlockSpec((1, 1, bn), lambda i: (i, 0, 0)),
            pl.BlockSpec((nb, d), lambda i: (0, 0)),
            pl.BlockSpec((nb, d), lambda i: (0, 0)),
            pl.BlockSpec((nb, d), lambda i: (0, 0)),
            pl.BlockSpec((1, d), lambda i: (0, 0)),
            pl.BlockSpec((1, d), lambda i: (0, 0)),
            pl.BlockSpec((1, d), lambda i: (0, 0)),
        ],
        out_specs=pl.BlockSpec((bn, d), lambda i: (i, 0)),
        out_shape=jax.ShapeDtypeStruct((n, d), jnp.float32),
    )(y, batch3, s0, s1, s2, alpha, gamma, beta)


def kernel(x, edge_index, edge_attr, u, batch, agg_w1, agg_b1, agg_w2, agg_b2,
           upd_w1, upd_b1, upd_w2, upd_b2, gn_alpha, gn_gamma, gn_beta):
    n, d = x.shape
    e = edge_index.shape[1]
    nb, gd = u.shape
    h = agg_w1.shape[1]
    hu = upd_w1.shape[1]

    src = edge_index[0]
    dst = edge_index[1]

    xs, cnts = _sc_gather_count(x, src, dst, n, e, d)

    msg = _edge_mlp(xs, edge_attr, agg_w1[:d], agg_w1[d:],
                    agg_b1.reshape(1, h), agg_w2, agg_b2.reshape(1, d),
                    e, d, h)

    parts = _sc_scatter(msg, dst, n, e, d)

    batch3 = batch.reshape(n // 1000, 1, 1000)
    y, s0, s1, s2 = _node_mlp(
        x, parts[0], parts[1], cnts[0], cnts[1], u, batch3,
        upd_w1[:d], upd_w1[d:2 * d], upd_w1[2 * d:],
        upd_b1.reshape(1, hu), upd_w2, upd_b2.reshape(1, d),
        n, d, nb, gd, hu)

    return _graphnorm(y, batch3, s0, s1, s2,
                      gn_alpha.reshape(1, d), gn_gamma.reshape(1, d),
                      gn_beta.reshape(1, d), n, d, nb)


# speedup vs baseline: 3.2946x; 1.2572x over previous
import functools

import jax
import jax.numpy as jnp
from jax import lax
from jax.experimental import pallas as pl
from jax.experimental.pallas import tpu as pltpu
from jax.experimental.pallas import tpu_sc as plsc

_NC = 2
_NS = 16
_NW = _NC * _NS
_K = 128
_NT = 632
_NPAD = _NT * _NS
_ZCHUNKS = ((0, 128), (128, 128), (256, 128), (384, 128), (512, 120))


def _sc_mesh():
    return plsc.VectorSubcoreMesh(
        core_axis_name="c", subcore_axis_name="s",
        num_cores=_NC, num_subcores=_NS)


def _sc_gather_count(x, src, dst, n, e, d):
    nch = e // _K
    consts = jnp.stack([jnp.zeros((_K, d), jnp.float32),
                        jnp.ones((_K, d), jnp.float32)])

    @functools.partial(
        pl.kernel,
        out_type=(jax.ShapeDtypeStruct((nch, _K, d), jnp.float32),
                  jax.ShapeDtypeStruct((_NC * _NPAD, d), jnp.float32)),
        mesh=_sc_mesh(),
        scratch_types=[
            pltpu.VMEM((_K,), jnp.int32),
            pltpu.VMEM((_K,), jnp.int32),
            pltpu.VMEM((_K,), jnp.int32),
            pltpu.VMEM((_K, d), jnp.float32),
            pltpu.VMEM((_K, d), jnp.float32),
            pltpu.VMEM((_K, d), jnp.float32),
            pltpu.VMEM_SHARED((_NPAD, d), jnp.float32),
            pltpu.SemaphoreType.DMA,
            pltpu.SemaphoreType.DMA,
        ],
    )
    def gk(x_hbm, src_hbm, dst_hbm, const_hbm, out_hbm, cnt_hbm,
           idxs0, idxs1, idxd, rows0, rows1, ones_v, cnt_sh, sem0, sem1):
        cid = lax.axis_index("c")
        sid = lax.axis_index("s")
        wid = cid * _NS + sid
        nj = nch // _NW + jnp.where(wid < nch % _NW, 1, 0)

        pltpu.sync_copy(const_hbm.at[0], rows0)
        pltpu.sync_copy(const_hbm.at[1], ones_v)

        for zo, zs in _ZCHUNKS:
            pltpu.sync_copy(rows0.at[pl.ds(0, zs)],
                            cnt_sh.at[pl.ds(sid * _NT + zo, zs)])
        plsc.subcore_barrier()

        idxs = (idxs0, idxs1)
        rows = (rows0, rows1)
        sems = (sem0, sem1)

        def chunk(j):
            return (wid + j * _NW) * _K

        def gather_start(j, b):
            pltpu.sync_copy(src_hbm.at[pl.ds(chunk(j), _K)], idxs[b])
            pltpu.async_copy(x_hbm.at[idxs[b]], rows[b], sems[b])

        def gather_wait(b):
            pltpu.make_async_copy(x_hbm.at[idxs[b]], rows[b], sems[b]).wait()

        @pl.when(nj > 0)
        def _():
            gather_start(0, 0)

        def pair(g, carry):
            for b in (0, 1):
                j = 2 * g + b

                @pl.when(j < nj)
                def _():
                    @pl.when(j + 1 < nj)
                    def _():
                        gather_start(j + 1, 1 - b)

                    gather_wait(b)
                    pltpu.sync_copy(rows[b], out_hbm.at[wid + j * _NW])
                    pltpu.sync_copy(dst_hbm.at[pl.ds(chunk(j), _K)], idxd)
                    pltpu.sync_copy(ones_v, cnt_sh.at[idxd], add=True)
            return carry

        lax.fori_loop(0, (nch // _NW + 2) // 2, pair, 0)
        plsc.subcore_barrier()

        for zo, zs in _ZCHUNKS:
            off = sid * _NT + zo
            pltpu.sync_copy(cnt_sh.at[pl.ds(off, zs)], rows0.at[pl.ds(0, zs)])
            pltpu.sync_copy(rows0.at[pl.ds(0, zs)],
                            cnt_hbm.at[pl.ds(cid * _NPAD + off, zs)])

    xs, cnt = gk(x, src, dst, consts)
    return xs.reshape(e, d), cnt.reshape(_NC, _NPAD, d)[:, :n]


def _sc_scatter(msg, dst, n, e, d):
    nch = e // _K
    msg3 = msg.reshape(nch, _K, d)
    zeros = jnp.zeros((_K, d), jnp.float32)

    @functools.partial(
        pl.kernel,
        out_type=jax.ShapeDtypeStruct((_NC * _NPAD, d), jnp.float32),
        mesh=_sc_mesh(),
        scratch_types=[
            pltpu.VMEM((_K,), jnp.int32),
            pltpu.VMEM((_K,), jnp.int32),
            pltpu.VMEM((_K, d), jnp.float32),
            pltpu.VMEM((_K, d), jnp.float32),
            pltpu.VMEM_SHARED((_NPAD, d), jnp.float32),
            pltpu.SemaphoreType.DMA,
            pltpu.SemaphoreType.DMA,
        ],
    )
    def sk(msg_hbm, dst_hbm, zero_hbm, out_hbm, idxd0, idxd1, buf0, buf1,
           acc_sh, sem0, sem1):
        cid = lax.axis_index("c")
        sid = lax.axis_index("s")
        wid = cid * _NS + sid
        nj = nch // _NW + jnp.where(wid < nch % _NW, 1, 0)

        pltpu.sync_copy(zero_hbm, buf0)

        for zo, zs in _ZCHUNKS:
            pltpu.sync_copy(buf0.at[pl.ds(0, zs)],
                            acc_sh.at[pl.ds(sid * _NT + zo, zs)])
        plsc.subcore_barrier()

        idxd = (idxd0, idxd1)
        bufs = (buf0, buf1)
        sems = (sem0, sem1)

        def msg_start(j, b):
            pltpu.sync_copy(dst_hbm.at[pl.ds((wid + j * _NW) * _K, _K)],
                            idxd[b])
            pltpu.async_copy(msg_hbm.at[wid + j * _NW], bufs[b], sems[b])

        def msg_wait(j, b):
            pltpu.make_async_copy(msg_hbm.at[wid + j * _NW], bufs[b],
                                  sems[b]).wait()

        @pl.when(nj > 0)
        def _():
            msg_start(0, 0)

        def pair(g, carry):
            for b in (0, 1):
                j = 2 * g + b

                @pl.when(j < nj)
                def _():
                    @pl.when(j + 1 < nj)
                    def _():
                        msg_start(j + 1, 1 - b)

                    msg_wait(j, b)
                    pltpu.sync_copy(bufs[b], acc_sh.at[idxd[b]], add=True)
            return carry

        lax.fori_loop(0, (nch // _NW + 2) // 2, pair, 0)
        plsc.subcore_barrier()

        for zo, zs in _ZCHUNKS:
            off = sid * _NT + zo
            pltpu.sync_copy(acc_sh.at[pl.ds(off, zs)], buf0.at[pl.ds(0, zs)])
            pltpu.sync_copy(buf0.at[pl.ds(0, zs)],
                            out_hbm.at[pl.ds(cid * _NPAD + off, zs)])

    return sk(msg3, dst, zeros).reshape(_NC, _NPAD, d)[:, :n]


def _edge_mlp(xs, ea, w1a, w1b, b1, w2, b2, e, d, h):
    be = 2000
    grid = e // be

    def body(xs_ref, ea_ref, w1a_ref, w1b_ref, b1_ref, w2_ref, b2_ref, o_ref):
        hv = (jnp.dot(xs_ref[...], w1a_ref[...], preferred_element_type=jnp.float32)
              + jnp.dot(ea_ref[...], w1b_ref[...], preferred_element_type=jnp.float32)
              + b1_ref[...])
        hv = jnp.maximum(hv, 0.0)
        o_ref[...] = (jnp.dot(hv, w2_ref[...], preferred_element_type=jnp.float32)
                      + b2_ref[...])

    return pl.pallas_call(
        body,
        grid=(grid,),
        in_specs=[
            pl.BlockSpec((be, d), lambda i: (i, 0)),
            pl.BlockSpec((be, d), lambda i: (i, 0)),
            pl.BlockSpec((d, h), lambda i: (0, 0)),
            pl.BlockSpec((d, h), lambda i: (0, 0)),
            pl.BlockSpec((1, h), lambda i: (0, 0)),
            pl.BlockSpec((h, d), lambda i: (0, 0)),
            pl.BlockSpec((1, d), lambda i: (0, 0)),
        ],
        out_specs=pl.BlockSpec((be, d), lambda i: (i, 0)),
        out_shape=jax.ShapeDtypeStruct((e, d), jnp.float32),
    )(xs, ea, w1a, w1b, b1, w2, b2)


def _node_mlp(x, p0, p1, c0, c1, u, batch3, w1a, w1b, w1c, b1, w2, b2,
              n, d, nb, gd, h):
    bn = 1000
    grid = n // bn

    def body(x_ref, p0_ref, p1_ref, c0_ref, c1_ref, u_ref, b_ref,
             w1a_ref, w1b_ref, w1c_ref, b1_ref, w2_ref, b2_ref,
             y_ref, s0_ref, s1_ref, s2_ref):
        i = pl.program_id(0)
        xv = x_ref[...]
        p = p0_ref[...] + p1_ref[...]
        cnt = jnp.max(c0_ref[...] + c1_ref[...], axis=1, keepdims=True)
        agg = p / jnp.clip(cnt, 1.0, None)
        bv = b_ref[0, 0, :]
        oh = (bv[:, None] == lax.broadcasted_iota(jnp.int32, (bn, nb), 1)
              ).astype(jnp.float32)
        ub = jnp.dot(oh, u_ref[...], preferred_element_type=jnp.float32)
        hv = (jnp.dot(xv, w1a_ref[...], preferred_element_type=jnp.float32)
              + jnp.dot(agg, w1b_ref[...], preferred_element_type=jnp.float32)
              + jnp.dot(ub, w1c_ref[...], preferred_element_type=jnp.float32)
              + b1_ref[...])
        hv = jnp.maximum(hv, 0.0)
        y = (jnp.dot(hv, w2_ref[...], preferred_element_type=jnp.float32)
             + b2_ref[...] + xv)
        y_ref[...] = y

        dg = lambda a, b: lax.dot_general(
            a, b, (((0,), (0,)), ((), ())), preferred_element_type=jnp.float32)

        @pl.when(i == 0)
        def _():
            s0_ref[...] = jnp.zeros_like(s0_ref)
            s1_ref[...] = jnp.zeros_like(s1_ref)
            s2_ref[...] = jnp.zeros_like(s2_ref)

        s0_ref[...] += dg(oh, jnp.ones_like(y))
        s1_ref[...] += dg(oh, y)
        s2_ref[...] += dg(oh, y * y)

    return pl.pallas_call(
        body,
        grid=(grid,),
        in_specs=[
            pl.BlockSpec((bn, d), lambda i: (i, 0)),
            pl.BlockSpec((bn, d), lambda i: (i, 0)),
            pl.BlockSpec((bn, d), lambda i: (i, 0)),
            pl.BlockSpec((bn, d), lambda i: (i, 0)),
            pl.BlockSpec((bn, d), lambda i: (i, 0)),
            pl.BlockSpec((nb, gd), lambda i: (0, 0)),
            pl.BlockSpec((1, 1, bn), lambda i: (i, 0, 0)),
            pl.BlockSpec((d, h), lambda i: (0, 0)),
            pl.BlockSpec((d, h), lambda i: (0, 0)),
            pl.BlockSpec((gd, h), lambda i: (0, 0)),
            pl.BlockSpec((1, h), lambda i: (0, 0)),
            pl.BlockSpec((h, d), lambda i: (0, 0)),
            pl.BlockSpec((1, d), lambda i: (0, 0)),
        ],
        out_specs=[
            pl.BlockSpec((bn, d), lambda i: (i, 0)),
            pl.BlockSpec((nb, d), lambda i: (0, 0)),
            pl.BlockSpec((nb, d), lambda i: (0, 0)),
            pl.BlockSpec((nb, d), lambda i: (0, 0)),
        ],
        out_shape=[
            jax.ShapeDtypeStruct((n, d), jnp.float32),
            jax.ShapeDtypeStruct((nb, d), jnp.float32),
            jax.ShapeDtypeStruct((nb, d), jnp.float32),
            jax.ShapeDtypeStruct((nb, d), jnp.float32),
        ],
    )(x, p0, p1, c0, c1, u, batch3, w1a, w1b, w1c, b1, w2, b2)


def _graphnorm(y, batch3, s0, s1, s2, alpha, gamma, beta, n, d, nb):
    bn = 1000
    grid = n // bn

    def body(y_ref, b_ref, s0_ref, s1_ref, s2_ref, al_ref, ga_ref, be_ref,
             o_ref):
        gcnt = jnp.clip(s0_ref[...], 1.0, None)
        mean = s1_ref[...] / gcnt
        am = al_ref[...] * mean
        var = s2_ref[...] / gcnt - am * (2.0 * mean - am)
        scale = ga_ref[...] / jnp.sqrt(var + 1e-5)
        shift = be_ref[...] - scale * am
        bv = b_ref[0, 0, :]
        oh = (bv[:, None] == lax.broadcasted_iota(jnp.int32, (bn, nb), 1)
              ).astype(jnp.float32)
        o_ref[...] = (jnp.dot(oh, scale, preferred_element_type=jnp.float32)
                      * y_ref[...]
                      + jnp.dot(oh, shift, preferred_element_type=jnp.float32))

    return pl.pallas_call(
        body,
        grid=(grid,),
        in_specs=[
            pl.BlockSpec((bn, d), lambda i: (i, 0)),
            pl.BlockSpec((1, 1, bn), lambda i: (i, 0, 0)),
            pl.BlockSpec((nb, d), lambda i: (0, 0)),
            pl.BlockSpec((nb, d), lambda i: (0, 0)),
            pl.BlockSpec((nb, d), lambda i: (0, 0)),
            pl.BlockSpec((1, d), lambda i: (0, 0)),
            pl.BlockSpec((1, d), lambda i: (0, 0)),
            pl.BlockSpec((1, d), lambda i: (0, 0)),
        ],
        out_specs=pl.BlockSpec((bn, d), lambda i: (i, 0)),
        out_shape=jax.ShapeDtypeStruct((n, d), jnp.float32),
    )(y, batch3, s0, s1, s2, alpha, gamma, beta)


def kernel(x, edge_index, edge_attr, u, batch, agg_w1, agg_b1, agg_w2, agg_b2,
           upd_w1, upd_b1, upd_w2, upd_b2, gn_alpha, gn_gamma, gn_beta):
    n, d = x.shape
    e = edge_index.shape[1]
    nb, gd = u.shape
    h = agg_w1.shape[1]
    hu = upd_w1.shape[1]

    src = edge_index[0]
    dst = edge_index[1]

    xs, cnts = _sc_gather_count(x, src, dst, n, e, d)

    msg = _edge_mlp(xs, edge_attr, agg_w1[:d], agg_w1[d:],
                    agg_b1.reshape(1, h), agg_w2, agg_b2.reshape(1, d),
                    e, d, h)

    parts = _sc_scatter(msg, dst, n, e, d)

    batch3 = batch.reshape(n // 1000, 1, 1000)
    y, s0, s1, s2 = _node_mlp(
        x, parts[0], parts[1], cnts[0], cnts[1], u, batch3,
        upd_w1[:d], upd_w1[d:2 * d], upd_w1[2 * d:],
        upd_b1.reshape(1, hu), upd_w2, upd_b2.reshape(1, d),
        n, d, nb, gd, hu)

    return _graphnorm(y, batch3, s0, s1, s2,
                      gn_alpha.reshape(1, d), gn_gamma.reshape(1, d),
                      gn_beta.reshape(1, d), n, d, nb)


# speedup vs baseline: 3.2956x; 1.0003x over previous
import functools

import jax
import jax.numpy as jnp
from jax import lax
from jax.experimental import pallas as pl
from jax.experimental.pallas import tpu as pltpu
from jax.experimental.pallas import tpu_sc as plsc

_NC = 2
_NS = 16
_NW = _NC * _NS
_K = 128
_NT = 632
_NPAD = _NT * _NS
_ZCHUNKS = ((0, 128), (128, 128), (256, 128), (384, 128), (512, 120))


def _sc_mesh():
    return plsc.VectorSubcoreMesh(
        core_axis_name="c", subcore_axis_name="s",
        num_cores=_NC, num_subcores=_NS)


def _sc_gather_count(x, src, dst, n, e, d):
    nch = e // _K
    consts = jnp.stack([jnp.zeros((_K, d), jnp.float32),
                        jnp.ones((_K, d), jnp.float32)])

    @functools.partial(
        pl.kernel,
        out_type=(jax.ShapeDtypeStruct((nch, _K, d), jnp.float32),
                  jax.ShapeDtypeStruct((_NC * _NPAD, d), jnp.float32)),
        mesh=_sc_mesh(),
        scratch_types=[
            pltpu.VMEM((_K,), jnp.int32),
            pltpu.VMEM((_K,), jnp.int32),
            pltpu.VMEM((_K,), jnp.int32),
            pltpu.VMEM((_K, d), jnp.float32),
            pltpu.VMEM((_K, d), jnp.float32),
            pltpu.VMEM((_K, d), jnp.float32),
            pltpu.VMEM_SHARED((_NPAD, d), jnp.float32),
            pltpu.SemaphoreType.DMA,
            pltpu.SemaphoreType.DMA,
        ],
    )
    def gk(x_hbm, src_hbm, dst_hbm, const_hbm, out_hbm, cnt_hbm,
           idxs0, idxs1, idxd, rows0, rows1, ones_v, cnt_sh, sem0, sem1):
        cid = lax.axis_index("c")
        sid = lax.axis_index("s")
        wid = cid * _NS + sid
        nj = nch // _NW + jnp.where(wid < nch % _NW, 1, 0)

        pltpu.sync_copy(const_hbm.at[0], rows0)
        pltpu.sync_copy(const_hbm.at[1], ones_v)

        for zo, zs in _ZCHUNKS:
            pltpu.sync_copy(rows0.at[pl.ds(0, zs)],
                            cnt_sh.at[pl.ds(sid * _NT + zo, zs)])
        plsc.subcore_barrier()

        idxs = (idxs0, idxs1)
        rows = (rows0, rows1)
        sems = (sem0, sem1)

        def chunk(j):
            return (wid + j * _NW) * _K

        def gather_start(j, b):
            pltpu.sync_copy(src_hbm.at[pl.ds(chunk(j), _K)], idxs[b])
            pltpu.async_copy(x_hbm.at[idxs[b]], rows[b], sems[b])

        def gather_wait(b):
            pltpu.make_async_copy(x_hbm.at[idxs[b]], rows[b], sems[b]).wait()

        @pl.when(nj > 0)
        def _():
            gather_start(0, 0)

        def pair(g, carry):
            for b in (0, 1):
                j = 2 * g + b

                @pl.when(j < nj)
                def _():
                    @pl.when(j + 1 < nj)
                    def _():
                        gather_start(j + 1, 1 - b)

                    gather_wait(b)
                    pltpu.sync_copy(rows[b], out_hbm.at[wid + j * _NW])
                    pltpu.sync_copy(dst_hbm.at[pl.ds(chunk(j), _K)], idxd)
                    pltpu.sync_copy(ones_v, cnt_sh.at[idxd], add=True)
            return carry

        lax.fori_loop(0, (nch // _NW + 2) // 2, pair, 0)
        plsc.subcore_barrier()

        for zo, zs in _ZCHUNKS:
            off = sid * _NT + zo
            pltpu.sync_copy(cnt_sh.at[pl.ds(off, zs)], rows0.at[pl.ds(0, zs)])
            pltpu.sync_copy(rows0.at[pl.ds(0, zs)],
                            cnt_hbm.at[pl.ds(cid * _NPAD + off, zs)])

    xs, cnt = gk(x, src, dst, consts)
    return xs.reshape(e, d), cnt.reshape(_NC, _NPAD, d)[:, :n]


def _sc_scatter(msg, dst, n, e, d):
    nch = e // _K
    msg3 = msg.reshape(nch, _K, d)
    zeros = jnp.zeros((_K, d), jnp.float32)

    @functools.partial(
        pl.kernel,
        out_type=jax.ShapeDtypeStruct((_NC * _NPAD, d), jnp.float32),
        mesh=_sc_mesh(),
        scratch_types=[
            pltpu.VMEM((_K,), jnp.int32),
            pltpu.VMEM((_K,), jnp.int32),
            pltpu.VMEM((_K, d), jnp.float32),
            pltpu.VMEM((_K, d), jnp.float32),
            pltpu.VMEM_SHARED((_NPAD, d), jnp.float32),
            pltpu.SemaphoreType.DMA,
            pltpu.SemaphoreType.DMA,
        ],
    )
    def sk(msg_hbm, dst_hbm, zero_hbm, out_hbm, idxd0, idxd1, buf0, buf1,
           acc_sh, sem0, sem1):
        cid = lax.axis_index("c")
        sid = lax.axis_index("s")
        wid = cid * _NS + sid
        nj = nch // _NW + jnp.where(wid < nch % _NW, 1, 0)

        pltpu.sync_copy(zero_hbm, buf0)

        for zo, zs in _ZCHUNKS:
            pltpu.sync_copy(buf0.at[pl.ds(0, zs)],
                            acc_sh.at[pl.ds(sid * _NT + zo, zs)])
        plsc.subcore_barrier()

        idxd = (idxd0, idxd1)
        bufs = (buf0, buf1)
        sems = (sem0, sem1)

        def msg_start(j, b):
            pltpu.sync_copy(dst_hbm.at[pl.ds((wid + j * _NW) * _K, _K)],
                            idxd[b])
            pltpu.async_copy(msg_hbm.at[wid + j * _NW], bufs[b], sems[b])

        def msg_wait(j, b):
            pltpu.make_async_copy(msg_hbm.at[wid + j * _NW], bufs[b],
                                  sems[b]).wait()

        @pl.when(nj > 0)
        def _():
            msg_start(0, 0)

        def pair(g, carry):
            for b in (0, 1):
                j = 2 * g + b

                @pl.when(j < nj)
                def _():
                    @pl.when(j + 1 < nj)
                    def _():
                        msg_start(j + 1, 1 - b)

                    msg_wait(j, b)
                    pltpu.sync_copy(bufs[b], acc_sh.at[idxd[b]], add=True)
            return carry

        lax.fori_loop(0, (nch // _NW + 2) // 2, pair, 0)
        plsc.subcore_barrier()

        for zo, zs in _ZCHUNKS:
            off = sid * _NT + zo
            pltpu.sync_copy(acc_sh.at[pl.ds(off, zs)], buf0.at[pl.ds(0, zs)])
            pltpu.sync_copy(buf0.at[pl.ds(0, zs)],
                            out_hbm.at[pl.ds(cid * _NPAD + off, zs)])

    return sk(msg3, dst, zeros).reshape(_NC, _NPAD, d)[:, :n]


def _edge_mlp(xs, ea, w1a, w1b, b1, w2, b2, e, d, h):
    be = 2000
    grid = e // be

    bf = jnp.bfloat16

    def body(xs_ref, ea_ref, w1a_ref, w1b_ref, b1_ref, w2_ref, b2_ref, o_ref):
        hv = (jnp.dot(xs_ref[...].astype(bf), w1a_ref[...],
                      preferred_element_type=jnp.float32)
              + jnp.dot(ea_ref[...].astype(bf), w1b_ref[...],
                        preferred_element_type=jnp.float32)
              + b1_ref[...])
        hv = jnp.maximum(hv, 0.0)
        o_ref[...] = (jnp.dot(hv.astype(bf), w2_ref[...],
                              preferred_element_type=jnp.float32)
                      + b2_ref[...])

    return pl.pallas_call(
        body,
        grid=(grid,),
        in_specs=[
            pl.BlockSpec((be, d), lambda i: (i, 0)),
            pl.BlockSpec((be, d), lambda i: (i, 0)),
            pl.BlockSpec((d, h), lambda i: (0, 0)),
            pl.BlockSpec((d, h), lambda i: (0, 0)),
            pl.BlockSpec((1, h), lambda i: (0, 0)),
            pl.BlockSpec((h, d), lambda i: (0, 0)),
            pl.BlockSpec((1, d), lambda i: (0, 0)),
        ],
        out_specs=pl.BlockSpec((be, d), lambda i: (i, 0)),
        out_shape=jax.ShapeDtypeStruct((e, d), jnp.float32),
    )(xs, ea, w1a, w1b, b1, w2, b2)


def _node_mlp(x, p0, p1, c0, c1, u, batch3, w1a, w1b, w1c, b1, w2, b2,
              n, d, nb, gd, h):
    bn = 1000
    grid = n // bn

    def body(x_ref, p0_ref, p1_ref, c0_ref, c1_ref, u_ref, b_ref,
             w1a_ref, w1b_ref, w1c_ref, b1_ref, w2_ref, b2_ref,
             y_ref, s0_ref, s1_ref, s2_ref):
        i = pl.program_id(0)
        xv = x_ref[...]
        p = p0_ref[...] + p1_ref[...]
        cnt = jnp.max(c0_ref[...] + c1_ref[...], axis=1, keepdims=True)
        agg = p / jnp.clip(cnt, 1.0, None)
        bv = b_ref[0, 0, :]
        oh = (bv[:, None] == lax.broadcasted_iota(jnp.int32, (bn, nb), 1)
              ).astype(jnp.float32)
        ub = jnp.dot(oh, u_ref[...], preferred_element_type=jnp.float32)
        hv = (jnp.dot(xv, w1a_ref[...], preferred_element_type=jnp.float32)
              + jnp.dot(agg, w1b_ref[...], preferred_element_type=jnp.float32)
              + jnp.dot(ub, w1c_ref[...], preferred_element_type=jnp.float32)
              + b1_ref[...])
        hv = jnp.maximum(hv, 0.0)
        y = (jnp.dot(hv, w2_ref[...], preferred_element_type=jnp.float32)
             + b2_ref[...] + xv)
        y_ref[...] = y

        dg = lambda a, b: lax.dot_general(
            a, b, (((0,), (0,)), ((), ())), preferred_element_type=jnp.float32)

        @pl.when(i == 0)
        def _():
            s0_ref[...] = jnp.zeros_like(s0_ref)
            s1_ref[...] = jnp.zeros_like(s1_ref)
            s2_ref[...] = jnp.zeros_like(s2_ref)

        s0_ref[...] += dg(oh, jnp.ones_like(y))
        s1_ref[...] += dg(oh, y)
        s2_ref[...] += dg(oh, y * y)

    return pl.pallas_call(
        body,
        grid=(grid,),
        in_specs=[
            pl.BlockSpec((bn, d), lambda i: (i, 0)),
            pl.BlockSpec((bn, d), lambda i: (i, 0)),
            pl.BlockSpec((bn, d), lambda i: (i, 0)),
            pl.BlockSpec((bn, d), lambda i: (i, 0)),
            pl.BlockSpec((bn, d), lambda i: (i, 0)),
            pl.BlockSpec((nb, gd), lambda i: (0, 0)),
            pl.BlockSpec((1, 1, bn), lambda i: (i, 0, 0)),
            pl.BlockSpec((d, h), lambda i: (0, 0)),
            pl.BlockSpec((d, h), lambda i: (0, 0)),
            pl.BlockSpec((gd, h), lambda i: (0, 0)),
            pl.BlockSpec((1, h), lambda i: (0, 0)),
            pl.BlockSpec((h, d), lambda i: (0, 0)),
            pl.BlockSpec((1, d), lambda i: (0, 0)),
        ],
        out_specs=[
            pl.BlockSpec((bn, d), lambda i: (i, 0)),
            pl.BlockSpec((nb, d), lambda i: (0, 0)),
            pl.BlockSpec((nb, d), lambda i: (0, 0)),
            pl.BlockSpec((nb, d), lambda i: (0, 0)),
        ],
        out_shape=[
            jax.ShapeDtypeStruct((n, d), jnp.float32),
            jax.ShapeDtypeStruct((nb, d), jnp.float32),
            jax.ShapeDtypeStruct((nb, d), jnp.float32),
            jax.ShapeDtypeStruct((nb, d), jnp.float32),
        ],
    )(x, p0, p1, c0, c1, u, batch3, w1a, w1b, w1c, b1, w2, b2)


def _graphnorm(y, batch3, s0, s1, s2, alpha, gamma, beta, n, d, nb):
    bn = 1000
    grid = n // bn

    def body(y_ref, b_ref, s0_ref, s1_ref, s2_ref, al_ref, ga_ref, be_ref,
             o_ref):
        gcnt = jnp.clip(s0_ref[...], 1.0, None)
        mean = s1_ref[...] / gcnt
        am = al_ref[...] * mean
        var = s2_ref[...] / gcnt - am * (2.0 * mean - am)
        scale = ga_ref[...] / jnp.sqrt(var + 1e-5)
        shift = be_ref[...] - scale * am
        bv = b_ref[0, 0, :]
        oh = (bv[:, None] == lax.broadcasted_iota(jnp.int32, (bn, nb), 1)
              ).astype(jnp.float32)
        o_ref[...] = (jnp.dot(oh, scale, preferred_element_type=jnp.float32)
                      * y_ref[...]
                      + jnp.dot(oh, shift, preferred_element_type=jnp.float32))

    return pl.pallas_call(
        body,
        grid=(grid,),
        in_specs=[
            pl.BlockSpec((bn, d), lambda i: (i, 0)),
            pl.BlockSpec((1, 1, bn), lambda i: (i, 0, 0)),
            pl.BlockSpec((nb, d), lambda i: (0, 0)),
            pl.BlockSpec((nb, d), lambda i: (0, 0)),
            pl.BlockSpec((nb, d), lambda i: (0, 0)),
            pl.BlockSpec((1, d), lambda i: (0, 0)),
            pl.BlockSpec((1, d), lambda i: (0, 0)),
            pl.BlockSpec((1, d), lambda i: (0, 0)),
        ],
        out_specs=pl.BlockSpec((bn, d), lambda i: (i, 0)),
        out_shape=jax.ShapeDtypeStruct((n, d), jnp.float32),
    )(y, batch3, s0, s1, s2, alpha, gamma, beta)


def kernel(x, edge_index, edge_attr, u, batch, agg_w1, agg_b1, agg_w2, agg_b2,
           upd_w1, upd_b1, upd_w2, upd_b2, gn_alpha, gn_gamma, gn_beta):
    n, d = x.shape
    e = edge_index.shape[1]
    nb, gd = u.shape
    h = agg_w1.shape[1]
    hu = upd_w1.shape[1]

    src = edge_index[0]
    dst = edge_index[1]

    xs, cnts = _sc_gather_count(x, src, dst, n, e, d)

    bf = jnp.bfloat16
    msg = _edge_mlp(xs, edge_attr, agg_w1[:d].astype(bf), agg_w1[d:].astype(bf),
                    agg_b1.reshape(1, h), agg_w2.astype(bf),
                    agg_b2.reshape(1, d), e, d, h)

    parts = _sc_scatter(msg, dst, n, e, d)

    batch3 = batch.reshape(n // 1000, 1, 1000)
    y, s0, s1, s2 = _node_mlp(
        x, parts[0], parts[1], cnts[0], cnts[1], u, batch3,
        upd_w1[:d], upd_w1[d:2 * d], upd_w1[2 * d:],
        upd_b1.reshape(1, hu), upd_w2, upd_b2.reshape(1, d),
        n, d, nb, gd, hu)

    return _graphnorm(y, batch3, s0, s1, s2,
                      gn_alpha.reshape(1, d), gn_gamma.reshape(1, d),
                      gn_beta.reshape(1, d), n, d, nb)


# speedup vs baseline: 3.5121x; 1.0657x over previous
import functools

import jax
import jax.numpy as jnp
from jax import lax
from jax.experimental import pallas as pl
from jax.experimental.pallas import tpu as pltpu
from jax.experimental.pallas import tpu_sc as plsc

_NC = 2
_NS = 16
_NW = _NC * _NS
_K = 128
_NT = 632
_NPAD = _NT * _NS
_ZCHUNKS = ((0, 128), (128, 128), (256, 128), (384, 128), (512, 120))


def _sc_mesh():
    return plsc.VectorSubcoreMesh(
        core_axis_name="c", subcore_axis_name="s",
        num_cores=_NC, num_subcores=_NS)


def _sc_gather_count(x, src, dst, n, e, d):
    nch = e // _K
    consts = jnp.stack([jnp.zeros((_K, d), jnp.float32),
                        jnp.ones((_K, d), jnp.float32)])

    @functools.partial(
        pl.kernel,
        out_type=(jax.ShapeDtypeStruct((nch, _K, d), jnp.float32),
                  jax.ShapeDtypeStruct((_NC * _NPAD, d), jnp.float32)),
        mesh=_sc_mesh(),
        scratch_types=[
            pltpu.VMEM((_K,), jnp.int32),
            pltpu.VMEM((_K,), jnp.int32),
            pltpu.VMEM((_K,), jnp.int32),
            pltpu.VMEM((_K, d), jnp.float32),
            pltpu.VMEM((_K, d), jnp.float32),
            pltpu.VMEM((_K, d), jnp.float32),
            pltpu.VMEM_SHARED((_NPAD, d), jnp.float32),
            pltpu.SemaphoreType.DMA,
            pltpu.SemaphoreType.DMA,
        ],
    )
    def gk(x_hbm, src_hbm, dst_hbm, const_hbm, out_hbm, cnt_hbm,
           idxs0, idxs1, idxd, rows0, rows1, ones_v, cnt_sh, sem0, sem1):
        cid = lax.axis_index("c")
        sid = lax.axis_index("s")
        wid = cid * _NS + sid
        nj = nch // _NW + jnp.where(wid < nch % _NW, 1, 0)

        pltpu.sync_copy(const_hbm.at[0], rows0)
        pltpu.sync_copy(const_hbm.at[1], ones_v)

        for zo, zs in _ZCHUNKS:
            pltpu.sync_copy(rows0.at[pl.ds(0, zs)],
                            cnt_sh.at[pl.ds(sid * _NT + zo, zs)])
        plsc.subcore_barrier()

        idxs = (idxs0, idxs1)
        rows = (rows0, rows1)
        sems = (sem0, sem1)

        def chunk(j):
            return (wid + j * _NW) * _K

        def gather_start(j, b):
            pltpu.sync_copy(src_hbm.at[pl.ds(chunk(j), _K)], idxs[b])
            pltpu.async_copy(x_hbm.at[idxs[b]], rows[b], sems[b])

        def gather_wait(b):
            pltpu.make_async_copy(x_hbm.at[idxs[b]], rows[b], sems[b]).wait()

        @pl.when(nj > 0)
        def _():
            gather_start(0, 0)

        def pair(g, carry):
            for b in (0, 1):
                j = 2 * g + b

                @pl.when(j < nj)
                def _():
                    @pl.when(j + 1 < nj)
                    def _():
                        gather_start(j + 1, 1 - b)

                    gather_wait(b)
                    pltpu.sync_copy(rows[b], out_hbm.at[wid + j * _NW])
                    pltpu.sync_copy(dst_hbm.at[pl.ds(chunk(j), _K)], idxd)
                    pltpu.sync_copy(ones_v, cnt_sh.at[idxd], add=True)
            return carry

        lax.fori_loop(0, (nch // _NW + 2) // 2, pair, 0)
        plsc.subcore_barrier()

        for zo, zs in _ZCHUNKS:
            off = sid * _NT + zo
            pltpu.sync_copy(cnt_sh.at[pl.ds(off, zs)], rows0.at[pl.ds(0, zs)])
            pltpu.sync_copy(rows0.at[pl.ds(0, zs)],
                            cnt_hbm.at[pl.ds(cid * _NPAD + off, zs)])

    xs, cnt = gk(x, src, dst, consts)
    return xs.reshape(e, d), cnt.reshape(_NC, _NPAD, d)[:, :n]


def _sc_scatter(msg, dst, n, e, d):
    nch = e // _K
    msg3 = msg.reshape(nch, _K, d)
    zeros = jnp.zeros((_K, d), jnp.float32)

    @functools.partial(
        pl.kernel,
        out_type=jax.ShapeDtypeStruct((_NC * _NPAD, d), jnp.float32),
        mesh=_sc_mesh(),
        scratch_types=[
            pltpu.VMEM((_K,), jnp.int32),
            pltpu.VMEM((_K,), jnp.int32),
            pltpu.VMEM((_K, d), jnp.float32),
            pltpu.VMEM((_K, d), jnp.float32),
            pltpu.VMEM_SHARED((_NPAD, d), jnp.float32),
            pltpu.SemaphoreType.DMA,
            pltpu.SemaphoreType.DMA,
        ],
    )
    def sk(msg_hbm, dst_hbm, zero_hbm, out_hbm, idxd0, idxd1, buf0, buf1,
           acc_sh, sem0, sem1):
        cid = lax.axis_index("c")
        sid = lax.axis_index("s")
        wid = cid * _NS + sid
        nj = nch // _NW + jnp.where(wid < nch % _NW, 1, 0)

        pltpu.sync_copy(zero_hbm, buf0)

        for zo, zs in _ZCHUNKS:
            pltpu.sync_copy(buf0.at[pl.ds(0, zs)],
                            acc_sh.at[pl.ds(sid * _NT + zo, zs)])
        plsc.subcore_barrier()

        idxd = (idxd0, idxd1)
        bufs = (buf0, buf1)
        sems = (sem0, sem1)

        def msg_start(j, b):
            pltpu.sync_copy(dst_hbm.at[pl.ds((wid + j * _NW) * _K, _K)],
                            idxd[b])
            pltpu.async_copy(msg_hbm.at[wid + j * _NW], bufs[b], sems[b])

        def msg_wait(j, b):
            pltpu.make_async_copy(msg_hbm.at[wid + j * _NW], bufs[b],
                                  sems[b]).wait()

        @pl.when(nj > 0)
        def _():
            msg_start(0, 0)

        def pair(g, carry):
            for b in (0, 1):
                j = 2 * g + b

                @pl.when(j < nj)
                def _():
                    @pl.when(j + 1 < nj)
                    def _():
                        msg_start(j + 1, 1 - b)

                    msg_wait(j, b)
                    pltpu.sync_copy(bufs[b], acc_sh.at[idxd[b]], add=True)
            return carry

        lax.fori_loop(0, (nch // _NW + 2) // 2, pair, 0)
        plsc.subcore_barrier()

        for zo, zs in _ZCHUNKS:
            off = sid * _NT + zo
            pltpu.sync_copy(acc_sh.at[pl.ds(off, zs)], buf0.at[pl.ds(0, zs)])
            pltpu.sync_copy(buf0.at[pl.ds(0, zs)],
                            out_hbm.at[pl.ds(cid * _NPAD + off, zs)])

    return sk(msg3, dst, zeros).reshape(_NC, _NPAD, d)[:, :n]


def _edge_mlp(xs, ea, w1a, w1b, b1, w2, b2, e, d, h):
    be = 2000
    grid = e // be

    bf = jnp.bfloat16

    def body(xs_ref, ea_ref, w1a_ref, w1b_ref, b1_ref, w2_ref, b2_ref, o_ref):
        cat = jnp.concatenate(
            [xs_ref[...].astype(bf), ea_ref[...].astype(bf)], axis=1)
        w1 = jnp.concatenate([w1a_ref[...], w1b_ref[...]], axis=0)
        hv = jnp.dot(cat, w1, preferred_element_type=jnp.float32) + b1_ref[...]
        hv = jnp.maximum(hv, 0.0)
        o_ref[...] = (jnp.dot(hv.astype(bf), w2_ref[...],
                              preferred_element_type=jnp.float32)
                      + b2_ref[...])

    return pl.pallas_call(
        body,
        grid=(grid,),
        in_specs=[
            pl.BlockSpec((be, d), lambda i: (i, 0)),
            pl.BlockSpec((be, d), lambda i: (i, 0)),
            pl.BlockSpec((d, h), lambda i: (0, 0)),
            pl.BlockSpec((d, h), lambda i: (0, 0)),
            pl.BlockSpec((1, h), lambda i: (0, 0)),
            pl.BlockSpec((h, d), lambda i: (0, 0)),
            pl.BlockSpec((1, d), lambda i: (0, 0)),
        ],
        out_specs=pl.BlockSpec((be, d), lambda i: (i, 0)),
        out_shape=jax.ShapeDtypeStruct((e, d), jnp.float32),
    )(xs, ea, w1a, w1b, b1, w2, b2)


def _node_mlp(x, p0, p1, c0, c1, u, batch3, w1a, w1b, w1c, b1, w2, b2,
              n, d, nb, gd, h):
    bn = 1000
    grid = n // bn

    def body(x_ref, p0_ref, p1_ref, c0_ref, c1_ref, u_ref, b_ref,
             w1a_ref, w1b_ref, w1c_ref, b1_ref, w2_ref, b2_ref,
             y_ref, s0_ref, s1_ref, s2_ref):
        i = pl.program_id(0)
        xv = x_ref[...]
        p = p0_ref[...] + p1_ref[...]
        cnt = jnp.max(c0_ref[...] + c1_ref[...], axis=1, keepdims=True)
        agg = p / jnp.clip(cnt, 1.0, None)
        bv = b_ref[0, 0, :]
        oh = (bv[:, None] == lax.broadcasted_iota(jnp.int32, (bn, nb), 1)
              ).astype(jnp.float32)
        ub = jnp.dot(oh, u_ref[...], preferred_element_type=jnp.float32)
        hv = (jnp.dot(xv, w1a_ref[...], preferred_element_type=jnp.float32)
              + jnp.dot(agg, w1b_ref[...], preferred_element_type=jnp.float32)
              + jnp.dot(ub, w1c_ref[...], preferred_element_type=jnp.float32)
              + b1_ref[...])
        hv = jnp.maximum(hv, 0.0)
        y = (jnp.dot(hv, w2_ref[...], preferred_element_type=jnp.float32)
             + b2_ref[...] + xv)
        y_ref[...] = y

        dg = lambda a, b: lax.dot_general(
            a, b, (((0,), (0,)), ((), ())), preferred_element_type=jnp.float32)

        @pl.when(i == 0)
        def _():
            s0_ref[...] = jnp.zeros_like(s0_ref)
            s1_ref[...] = jnp.zeros_like(s1_ref)
            s2_ref[...] = jnp.zeros_like(s2_ref)

        s0_ref[...] += dg(oh, jnp.ones_like(y))
        s1_ref[...] += dg(oh, y)
        s2_ref[...] += dg(oh, y * y)

    return pl.pallas_call(
        body,
        grid=(grid,),
        in_specs=[
            pl.BlockSpec((bn, d), lambda i: (i, 0)),
            pl.BlockSpec((bn, d), lambda i: (i, 0)),
            pl.BlockSpec((bn, d), lambda i: (i, 0)),
            pl.BlockSpec((bn, d), lambda i: (i, 0)),
            pl.BlockSpec((bn, d), lambda i: (i, 0)),
            pl.BlockSpec((nb, gd), lambda i: (0, 0)),
            pl.BlockSpec((1, 1, bn), lambda i: (i, 0, 0)),
            pl.BlockSpec((d, h), lambda i: (0, 0)),
            pl.BlockSpec((d, h), lambda i: (0, 0)),
            pl.BlockSpec((gd, h), lambda i: (0, 0)),
            pl.BlockSpec((1, h), lambda i: (0, 0)),
            pl.BlockSpec((h, d), lambda i: (0, 0)),
            pl.BlockSpec((1, d), lambda i: (0, 0)),
        ],
        out_specs=[
            pl.BlockSpec((bn, d), lambda i: (i, 0)),
            pl.BlockSpec((nb, d), lambda i: (0, 0)),
            pl.BlockSpec((nb, d), lambda i: (0, 0)),
            pl.BlockSpec((nb, d), lambda i: (0, 0)),
        ],
        out_shape=[
            jax.ShapeDtypeStruct((n, d), jnp.float32),
            jax.ShapeDtypeStruct((nb, d), jnp.float32),
            jax.ShapeDtypeStruct((nb, d), jnp.float32),
            jax.ShapeDtypeStruct((nb, d), jnp.float32),
        ],
    )(x, p0, p1, c0, c1, u, batch3, w1a, w1b, w1c, b1, w2, b2)


def _graphnorm(y, batch3, s0, s1, s2, alpha, gamma, beta, n, d, nb):
    bn = 1000
    grid = n // bn

    def body(y_ref, b_ref, s0_ref, s1_ref, s2_ref, al_ref, ga_ref, be_ref,
             o_ref):
        gcnt = jnp.clip(s0_ref[...], 1.0, None)
        mean = s1_ref[...] / gcnt
        am = al_ref[...] * mean
        var = s2_ref[...] / gcnt - am * (2.0 * mean - am)
        scale = ga_ref[...] / jnp.sqrt(var + 1e-5)
        shift = be_ref[...] - scale * am
        bv = b_ref[0, 0, :]
        oh = (bv[:, None] == lax.broadcasted_iota(jnp.int32, (bn, nb), 1)
              ).astype(jnp.float32)
        o_ref[...] = (jnp.dot(oh, scale, preferred_element_type=jnp.float32)
                      * y_ref[...]
                      + jnp.dot(oh, shift, preferred_element_type=jnp.float32))

    return pl.pallas_call(
        body,
        grid=(grid,),
        in_specs=[
            pl.BlockSpec((bn, d), lambda i: (i, 0)),
            pl.BlockSpec((1, 1, bn), lambda i: (i, 0, 0)),
            pl.BlockSpec((nb, d), lambda i: (0, 0)),
            pl.BlockSpec((nb, d), lambda i: (0, 0)),
            pl.BlockSpec((nb, d), lambda i: (0, 0)),
            pl.BlockSpec((1, d), lambda i: (0, 0)),
            pl.BlockSpec((1, d), lambda i: (0, 0)),
            pl.BlockSpec((1, d), lambda i: (0, 0)),
        ],
        out_specs=pl.BlockSpec((bn, d), lambda i: (i, 0)),
        out_shape=jax.ShapeDtypeStruct((n, d), jnp.float32),
    )(y, batch3, s0, s1, s2, alpha, gamma, beta)


def kernel(x, edge_index, edge_attr, u, batch, agg_w1, agg_b1, agg_w2, agg_b2,
           upd_w1, upd_b1, upd_w2, upd_b2, gn_alpha, gn_gamma, gn_beta):
    n, d = x.shape
    e = edge_index.shape[1]
    nb, gd = u.shape
    h = agg_w1.shape[1]
    hu = upd_w1.shape[1]

    src = edge_index[0]
    dst = edge_index[1]

    xs, cnts = _sc_gather_count(x, src, dst, n, e, d)

    bf = jnp.bfloat16
    msg = _edge_mlp(xs, edge_attr, agg_w1[:d].astype(bf), agg_w1[d:].astype(bf),
                    agg_b1.reshape(1, h), agg_w2.astype(bf),
                    agg_b2.reshape(1, d), e, d, h)

    parts = _sc_scatter(msg, dst, n, e, d)

    batch3 = batch.reshape(n // 1000, 1, 1000)
    y, s0, s1, s2 = _node_mlp(
        x, parts[0], parts[1], cnts[0], cnts[1], u, batch3,
        upd_w1[:d], upd_w1[d:2 * d], upd_w1[2 * d:],
        upd_b1.reshape(1, hu), upd_w2, upd_b2.reshape(1, d),
        n, d, nb, gd, hu)

    return _graphnorm(y, batch3, s0, s1, s2,
                      gn_alpha.reshape(1, d), gn_gamma.reshape(1, d),
                      gn_beta.reshape(1, d), n, d, nb)


# speedup vs baseline: 3.8200x; 1.0877x over previous
import functools

import jax
import jax.numpy as jnp
from jax import lax
from jax.experimental import pallas as pl
from jax.experimental.pallas import tpu as pltpu
from jax.experimental.pallas import tpu_sc as plsc

_NC = 2
_NS = 16
_NW = _NC * _NS
_K = 128
_NT = 632
_NPAD = _NT * _NS
_ZCHUNKS = ((0, 128), (128, 128), (256, 128), (384, 128), (512, 120))


_MAXJ = 79


def _sc_mesh():
    return plsc.VectorSubcoreMesh(
        core_axis_name="c", subcore_axis_name="s",
        num_cores=_NC, num_subcores=_NS)


def _worker_range(wid, nch):
    base = nch // _NW
    ext = nch % _NW
    s = base * wid + jnp.minimum(wid, ext)
    nj = base + jnp.where(wid < ext, 1, 0)
    start = jnp.minimum(s, nch - _MAXJ)
    sh = s - start
    return s, nj, start, sh


def _sc_gather_count(x, src, dst, n, e, d):
    nch = e // _K
    consts = jnp.stack([jnp.zeros((_K, d), jnp.float32),
                        jnp.ones((_K, d), jnp.float32)])

    @functools.partial(
        pl.kernel,
        out_type=(jax.ShapeDtypeStruct((nch, _K, d), jnp.float32),
                  jax.ShapeDtypeStruct((_NC * _NPAD, d), jnp.float32)),
        mesh=_sc_mesh(),
        scratch_types=[
            pltpu.VMEM((_K,), jnp.int32),
            pltpu.VMEM((_K,), jnp.int32),
            pltpu.VMEM((_K,), jnp.int32),
            pltpu.VMEM((_K,), jnp.int32),
            pltpu.VMEM((_K, d), jnp.float32),
            pltpu.VMEM((_K, d), jnp.float32),
            pltpu.VMEM((_K, d), jnp.float32),
            pltpu.VMEM_SHARED((_NPAD, d), jnp.float32),
            pltpu.SemaphoreType.DMA,
            pltpu.SemaphoreType.DMA,
            pltpu.SemaphoreType.DMA,
            pltpu.SemaphoreType.DMA,
            pltpu.SemaphoreType.DMA,
            pltpu.SemaphoreType.DMA,
        ],
    )
    def gk(x_hbm, src_hbm, dst_hbm, const_hbm, out_hbm, cnt_hbm,
           idxs0, idxs1, idxd0, idxd1, rows0, rows1, ones_v, cnt_sh,
           sem0, sem1, dsem0, dsem1, isem0, isem1):
        cid = lax.axis_index("c")
        sid = lax.axis_index("s")
        wid = cid * _NS + sid
        s, nj, start, sh = _worker_range(wid, nch)

        pltpu.sync_copy(const_hbm.at[0], rows0)
        pltpu.sync_copy(const_hbm.at[1], ones_v)

        for zo, zs in _ZCHUNKS:
            pltpu.sync_copy(rows0.at[pl.ds(0, zs)],
                            cnt_sh.at[pl.ds(sid * _NT + zo, zs)])
        plsc.subcore_barrier()

        idxs = (idxs0, idxs1)
        idxd = (idxd0, idxd1)
        rows = (rows0, rows1)
        sems = (sem0, sem1)
        dsems = (dsem0, dsem1)
        isems = (isem0, isem1)

        def sidx_start(j, b):
            pltpu.async_copy(src_hbm.at[pl.ds((s + j) * _K, _K)], idxs[b],
                             isems[b])

        def sidx_wait(j, b):
            pltpu.make_async_copy(src_hbm.at[pl.ds((s + j) * _K, _K)],
                                  idxs[b], isems[b]).wait()

        def gather_start(j, b):
            pltpu.async_copy(x_hbm.at[idxs[b]], rows[b], sems[b])

        def gather_wait(j, b):
            pltpu.make_async_copy(x_hbm.at[idxs[b]], rows[b], sems[b]).wait()

        def didx_start(j, b):
            pltpu.async_copy(dst_hbm.at[pl.ds((s + j) * _K, _K)], idxd[b],
                             dsems[b])

        def didx_wait(j, b):
            pltpu.make_async_copy(dst_hbm.at[pl.ds((s + j) * _K, _K)],
                                  idxd[b], dsems[b]).wait()

        @pl.when(nj > 0)
        def _():
            pltpu.sync_copy(src_hbm.at[pl.ds(s * _K, _K)], idxs0)
            gather_start(0, 0)
            didx_start(0, 0)

        @pl.when(nj > 1)
        def _():
            sidx_start(1, 1)

        def pair(g, carry):
            for b in (0, 1):
                j = 2 * g + b

                @pl.when(j < nj)
                def _():
                    @pl.when(j + 1 < nj)
                    def _():
                        sidx_wait(j + 1, 1 - b)
                        gather_start(j + 1, 1 - b)
                        didx_start(j + 1, 1 - b)

                    gather_wait(j, b)
                    pltpu.sync_copy(rows[b], out_hbm.at[s + j])
                    didx_wait(j, b)
                    pltpu.sync_copy(ones_v, cnt_sh.at[idxd[b]], add=True)

                    @pl.when(j + 2 < nj)
                    def _():
                        sidx_start(j + 2, b)
            return carry

        lax.fori_loop(0, (nch // _NW + 2) // 2, pair, 0)
        plsc.subcore_barrier()

        for zo, zs in _ZCHUNKS:
            off = sid * _NT + zo
            pltpu.sync_copy(cnt_sh.at[pl.ds(off, zs)], rows0.at[pl.ds(0, zs)])
            pltpu.sync_copy(rows0.at[pl.ds(0, zs)],
                            cnt_hbm.at[pl.ds(cid * _NPAD + off, zs)])

    xs, cnt = gk(x, src, dst, consts)
    return xs.reshape(e, d), cnt.reshape(_NC, _NPAD, d)[:, :n]


def _sc_scatter(msg, dst, n, e, d):
    nch = e // _K
    msg3 = msg.reshape(nch, _K, d)
    zeros = jnp.zeros((_K, d), jnp.float32)

    @functools.partial(
        pl.kernel,
        out_type=jax.ShapeDtypeStruct((_NC * _NPAD, d), jnp.float32),
        mesh=_sc_mesh(),
        scratch_types=[
            pltpu.VMEM((_K,), jnp.int32),
            pltpu.VMEM((_K,), jnp.int32),
            pltpu.VMEM((_K, d), jnp.float32),
            pltpu.VMEM((_K, d), jnp.float32),
            pltpu.VMEM_SHARED((_NPAD, d), jnp.float32),
            pltpu.SemaphoreType.DMA,
            pltpu.SemaphoreType.DMA,
            pltpu.SemaphoreType.DMA,
            pltpu.SemaphoreType.DMA,
        ],
    )
    def sk(msg_hbm, dst_hbm, zero_hbm, out_hbm, idxd0, idxd1, buf0, buf1,
           acc_sh, sem0, sem1, dsem0, dsem1):
        cid = lax.axis_index("c")
        sid = lax.axis_index("s")
        wid = cid * _NS + sid
        s, nj, start, sh = _worker_range(wid, nch)

        pltpu.sync_copy(zero_hbm, buf0)

        for zo, zs in _ZCHUNKS:
            pltpu.sync_copy(buf0.at[pl.ds(0, zs)],
                            acc_sh.at[pl.ds(sid * _NT + zo, zs)])
        plsc.subcore_barrier()

        idxd = (idxd0, idxd1)
        bufs = (buf0, buf1)
        sems = (sem0, sem1)
        dsems = (dsem0, dsem1)

        def msg_start(j, b):
            pltpu.async_copy(msg_hbm.at[s + j], bufs[b], sems[b])
            pltpu.async_copy(dst_hbm.at[pl.ds((s + j) * _K, _K)], idxd[b],
                             dsems[b])

        def msg_wait(j, b):
            pltpu.make_async_copy(msg_hbm.at[s + j], bufs[b],
                                  sems[b]).wait()
            pltpu.make_async_copy(dst_hbm.at[pl.ds((s + j) * _K, _K)],
                                  idxd[b], dsems[b]).wait()

        @pl.when(nj > 0)
        def _():
            msg_start(0, 0)

        def pair(g, carry):
            for b in (0, 1):
                j = 2 * g + b

                @pl.when(j < nj)
                def _():
                    @pl.when(j + 1 < nj)
                    def _():
                        msg_start(j + 1, 1 - b)

                    msg_wait(j, b)
                    pltpu.sync_copy(bufs[b], acc_sh.at[idxd[b]], add=True)
            return carry

        lax.fori_loop(0, (nch // _NW + 2) // 2, pair, 0)
        plsc.subcore_barrier()

        for zo, zs in _ZCHUNKS:
            off = sid * _NT + zo
            pltpu.sync_copy(acc_sh.at[pl.ds(off, zs)], buf0.at[pl.ds(0, zs)])
            pltpu.sync_copy(buf0.at[pl.ds(0, zs)],
                            out_hbm.at[pl.ds(cid * _NPAD + off, zs)])

    return sk(msg3, dst, zeros).reshape(_NC, _NPAD, d)[:, :n]


def _edge_mlp(xs, ea, w1a, w1b, b1, w2, b2, e, d, h):
    be = 2000
    grid = e // be

    bf = jnp.bfloat16

    def body(xs_ref, ea_ref, w1a_ref, w1b_ref, b1_ref, w2_ref, b2_ref, o_ref):
        cat = jnp.concatenate(
            [xs_ref[...].astype(bf), ea_ref[...].astype(bf)], axis=1)
        w1 = jnp.concatenate([w1a_ref[...], w1b_ref[...]], axis=0)
        hv = jnp.dot(cat, w1, preferred_element_type=jnp.float32) + b1_ref[...]
        hv = jnp.maximum(hv, 0.0)
        o_ref[...] = (jnp.dot(hv.astype(bf), w2_ref[...],
                              preferred_element_type=jnp.float32)
                      + b2_ref[...])

    return pl.pallas_call(
        body,
        grid=(grid,),
        in_specs=[
            pl.BlockSpec((be, d), lambda i: (i, 0)),
            pl.BlockSpec((be, d), lambda i: (i, 0)),
            pl.BlockSpec((d, h), lambda i: (0, 0)),
            pl.BlockSpec((d, h), lambda i: (0, 0)),
            pl.BlockSpec((1, h), lambda i: (0, 0)),
            pl.BlockSpec((h, d), lambda i: (0, 0)),
            pl.BlockSpec((1, d), lambda i: (0, 0)),
        ],
        out_specs=pl.BlockSpec((be, d), lambda i: (i, 0)),
        out_shape=jax.ShapeDtypeStruct((e, d), jnp.float32),
    )(xs, ea, w1a, w1b, b1, w2, b2)


def _node_mlp(x, p0, p1, c0, c1, u, batch3, w1a, w1b, w1c, b1, w2, b2,
              n, d, nb, gd, h):
    bn = 1000
    grid = n // bn

    def body(x_ref, p0_ref, p1_ref, c0_ref, c1_ref, u_ref, b_ref,
             w1a_ref, w1b_ref, w1c_ref, b1_ref, w2_ref, b2_ref,
             y_ref, s0_ref, s1_ref, s2_ref):
        i = pl.program_id(0)
        xv = x_ref[...]
        p = p0_ref[...] + p1_ref[...]
        cnt = jnp.max(c0_ref[...] + c1_ref[...], axis=1, keepdims=True)
        agg = p / jnp.clip(cnt, 1.0, None)
        bv = b_ref[0, 0, :]
        oh = (bv[:, None] == lax.broadcasted_iota(jnp.int32, (bn, nb), 1)
              ).astype(jnp.float32)
        ub = jnp.dot(oh, u_ref[...], preferred_element_type=jnp.float32)
        hv = (jnp.dot(xv, w1a_ref[...], preferred_element_type=jnp.float32)
              + jnp.dot(agg, w1b_ref[...], preferred_element_type=jnp.float32)
              + jnp.dot(ub, w1c_ref[...], preferred_element_type=jnp.float32)
              + b1_ref[...])
        hv = jnp.maximum(hv, 0.0)
        y = (jnp.dot(hv, w2_ref[...], preferred_element_type=jnp.float32)
             + b2_ref[...] + xv)
        y_ref[...] = y

        dg = lambda a, b: lax.dot_general(
            a, b, (((0,), (0,)), ((), ())), preferred_element_type=jnp.float32)

        @pl.when(i == 0)
        def _():
            s0_ref[...] = jnp.zeros_like(s0_ref)
            s1_ref[...] = jnp.zeros_like(s1_ref)
            s2_ref[...] = jnp.zeros_like(s2_ref)

        s0_ref[...] += dg(oh, jnp.ones_like(y))
        s1_ref[...] += dg(oh, y)
        s2_ref[...] += dg(oh, y * y)

    return pl.pallas_call(
        body,
        grid=(grid,),
        in_specs=[
            pl.BlockSpec((bn, d), lambda i: (i, 0)),
            pl.BlockSpec((bn, d), lambda i: (i, 0)),
            pl.BlockSpec((bn, d), lambda i: (i, 0)),
            pl.BlockSpec((bn, d), lambda i: (i, 0)),
            pl.BlockSpec((bn, d), lambda i: (i, 0)),
            pl.BlockSpec((nb, gd), lambda i: (0, 0)),
            pl.BlockSpec((1, 1, bn), lambda i: (i, 0, 0)),
            pl.BlockSpec((d, h), lambda i: (0, 0)),
            pl.BlockSpec((d, h), lambda i: (0, 0)),
            pl.BlockSpec((gd, h), lambda i: (0, 0)),
            pl.BlockSpec((1, h), lambda i: (0, 0)),
            pl.BlockSpec((h, d), lambda i: (0, 0)),
            pl.BlockSpec((1, d), lambda i: (0, 0)),
        ],
        out_specs=[
            pl.BlockSpec((bn, d), lambda i: (i, 0)),
            pl.BlockSpec((nb, d), lambda i: (0, 0)),
            pl.BlockSpec((nb, d), lambda i: (0, 0)),
            pl.BlockSpec((nb, d), lambda i: (0, 0)),
        ],
        out_shape=[
            jax.ShapeDtypeStruct((n, d), jnp.float32),
            jax.ShapeDtypeStruct((nb, d), jnp.float32),
            jax.ShapeDtypeStruct((nb, d), jnp.float32),
            jax.ShapeDtypeStruct((nb, d), jnp.float32),
        ],
    )(x, p0, p1, c0, c1, u, batch3, w1a, w1b, w1c, b1, w2, b2)


def _graphnorm(y, batch3, s0, s1, s2, alpha, gamma, beta, n, d, nb):
    bn = 1000
    grid = n // bn

    def body(y_ref, b_ref, s0_ref, s1_ref, s2_ref, al_ref, ga_ref, be_ref,
             o_ref):
        gcnt = jnp.clip(s0_ref[...], 1.0, None)
        mean = s1_ref[...] / gcnt
        am = al_ref[...] * mean
        var = s2_ref[...] / gcnt - am * (2.0 * mean - am)
        scale = ga_ref[...] / jnp.sqrt(var + 1e-5)
        shift = be_ref[...] - scale * am
        bv = b_ref[0, 0, :]
        oh = (bv[:, None] == lax.broadcasted_iota(jnp.int32, (bn, nb), 1)
              ).astype(jnp.float32)
        o_ref[...] = (jnp.dot(oh, scale, preferred_element_type=jnp.float32)
                      * y_ref[...]
                      + jnp.dot(oh, shift, preferred_element_type=jnp.float32))

    return pl.pallas_call(
        body,
        grid=(grid,),
        in_specs=[
            pl.BlockSpec((bn, d), lambda i: (i, 0)),
            pl.BlockSpec((1, 1, bn), lambda i: (i, 0, 0)),
            pl.BlockSpec((nb, d), lambda i: (0, 0)),
            pl.BlockSpec((nb, d), lambda i: (0, 0)),
            pl.BlockSpec((nb, d), lambda i: (0, 0)),
            pl.BlockSpec((1, d), lambda i: (0, 0)),
            pl.BlockSpec((1, d), lambda i: (0, 0)),
            pl.BlockSpec((1, d), lambda i: (0, 0)),
        ],
        out_specs=pl.BlockSpec((bn, d), lambda i: (i, 0)),
        out_shape=jax.ShapeDtypeStruct((n, d), jnp.float32),
    )(y, batch3, s0, s1, s2, alpha, gamma, beta)


def kernel(x, edge_index, edge_attr, u, batch, agg_w1, agg_b1, agg_w2, agg_b2,
           upd_w1, upd_b1, upd_w2, upd_b2, gn_alpha, gn_gamma, gn_beta):
    n, d = x.shape
    e = edge_index.shape[1]
    nb, gd = u.shape
    h = agg_w1.shape[1]
    hu = upd_w1.shape[1]

    src = edge_index[0]
    dst = edge_index[1]

    xs, cnts = _sc_gather_count(x, src, dst, n, e, d)

    bf = jnp.bfloat16
    msg = _edge_mlp(xs, edge_attr, agg_w1[:d].astype(bf), agg_w1[d:].astype(bf),
                    agg_b1.reshape(1, h), agg_w2.astype(bf),
                    agg_b2.reshape(1, d), e, d, h)

    parts = _sc_scatter(msg, dst, n, e, d)

    batch3 = batch.reshape(n // 1000, 1, 1000)
    y, s0, s1, s2 = _node_mlp(
        x, parts[0], parts[1], cnts[0], cnts[1], u, batch3,
        upd_w1[:d], upd_w1[d:2 * d], upd_w1[2 * d:],
        upd_b1.reshape(1, hu), upd_w2, upd_b2.reshape(1, d),
        n, d, nb, gd, hu)

    return _graphnorm(y, batch3, s0, s1, s2,
                      gn_alpha.reshape(1, d), gn_gamma.reshape(1, d),
                      gn_beta.reshape(1, d), n, d, nb)


# speedup vs baseline: 4.1759x; 1.0932x over previous
import functools

import jax
import jax.numpy as jnp
from jax import lax
from jax.experimental import pallas as pl
from jax.experimental.pallas import tpu as pltpu
from jax.experimental.pallas import tpu_sc as plsc

_NC = 2
_NS = 16
_NW = _NC * _NS
_K = 128
_NT = 632
_NPAD = _NT * _NS
_ZCHUNKS = ((0, 128), (128, 128), (256, 128), (384, 128), (512, 120))


_MAXJ = 79


def _sc_mesh():
    return plsc.VectorSubcoreMesh(
        core_axis_name="c", subcore_axis_name="s",
        num_cores=_NC, num_subcores=_NS)


def _worker_range(wid, nch):
    base = nch // _NW
    ext = nch % _NW
    s = base * wid + jnp.minimum(wid, ext)
    nj = base + jnp.where(wid < ext, 1, 0)
    start = jnp.minimum(s, nch - _MAXJ)
    sh = s - start
    return s, nj, start, sh


def _sc_gather_count(x, src, dst, n, e, d):
    nch = e // _K
    consts = jnp.stack([jnp.zeros((_K, d), jnp.float32),
                        jnp.ones((_K, d), jnp.float32)])

    @functools.partial(
        pl.kernel,
        out_type=(jax.ShapeDtypeStruct((nch, _K, d), jnp.float32),
                  jax.ShapeDtypeStruct((_NC * _NPAD, d), jnp.float32)),
        mesh=_sc_mesh(),
        scratch_types=[
            pltpu.VMEM((_K,), jnp.int32),
            pltpu.VMEM((_K,), jnp.int32),
            pltpu.VMEM((_K,), jnp.int32),
            pltpu.VMEM((_K,), jnp.int32),
            pltpu.VMEM((_K, d), jnp.float32),
            pltpu.VMEM((_K, d), jnp.float32),
            pltpu.VMEM((_K, d), jnp.float32),
            pltpu.VMEM_SHARED((_NPAD, d), jnp.float32),
            pltpu.SemaphoreType.DMA,
            pltpu.SemaphoreType.DMA,
            pltpu.SemaphoreType.DMA,
            pltpu.SemaphoreType.DMA,
            pltpu.SemaphoreType.DMA,
            pltpu.SemaphoreType.DMA,
        ],
    )
    def gk(x_hbm, src_hbm, dst_hbm, const_hbm, out_hbm, cnt_hbm,
           idxs0, idxs1, idxd0, idxd1, rows0, rows1, ones_v, cnt_sh,
           sem0, sem1, dsem0, dsem1, isem0, isem1):
        cid = lax.axis_index("c")
        sid = lax.axis_index("s")
        wid = cid * _NS + sid
        s, nj, start, sh = _worker_range(wid, nch)

        pltpu.sync_copy(const_hbm.at[0], rows0)
        pltpu.sync_copy(const_hbm.at[1], ones_v)

        for zo, zs in _ZCHUNKS:
            pltpu.sync_copy(rows0.at[pl.ds(0, zs)],
                            cnt_sh.at[pl.ds(sid * _NT + zo, zs)])
        plsc.subcore_barrier()

        idxs = (idxs0, idxs1)
        idxd = (idxd0, idxd1)
        rows = (rows0, rows1)
        sems = (sem0, sem1)
        dsems = (dsem0, dsem1)
        isems = (isem0, isem1)

        def sidx_start(j, b):
            pltpu.async_copy(src_hbm.at[pl.ds((s + j) * _K, _K)], idxs[b],
                             isems[b])

        def sidx_wait(j, b):
            pltpu.make_async_copy(src_hbm.at[pl.ds((s + j) * _K, _K)],
                                  idxs[b], isems[b]).wait()

        def gather_start(j, b):
            pltpu.async_copy(x_hbm.at[idxs[b]], rows[b], sems[b])

        def gather_wait(j, b):
            pltpu.make_async_copy(x_hbm.at[idxs[b]], rows[b], sems[b]).wait()

        def didx_start(j, b):
            pltpu.async_copy(dst_hbm.at[pl.ds((s + j) * _K, _K)], idxd[b],
                             dsems[b])

        def didx_wait(j, b):
            pltpu.make_async_copy(dst_hbm.at[pl.ds((s + j) * _K, _K)],
                                  idxd[b], dsems[b]).wait()

        @pl.when(nj > 0)
        def _():
            pltpu.sync_copy(src_hbm.at[pl.ds(s * _K, _K)], idxs0)
            gather_start(0, 0)
            didx_start(0, 0)

        @pl.when(nj > 1)
        def _():
            sidx_start(1, 1)

        def pair(g, carry):
            for b in (0, 1):
                j = 2 * g + b

                @pl.when(j < nj)
                def _():
                    @pl.when(j + 1 < nj)
                    def _():
                        sidx_wait(j + 1, 1 - b)
                        gather_start(j + 1, 1 - b)
                        didx_start(j + 1, 1 - b)

                    gather_wait(j, b)
                    pltpu.sync_copy(rows[b], out_hbm.at[s + j])
                    didx_wait(j, b)
                    pltpu.sync_copy(ones_v, cnt_sh.at[idxd[b]], add=True)

                    @pl.when(j + 2 < nj)
                    def _():
                        sidx_start(j + 2, b)
            return carry

        lax.fori_loop(0, (nch // _NW + 2) // 2, pair, 0)
        plsc.subcore_barrier()

        for zo, zs in _ZCHUNKS:
            off = sid * _NT + zo
            pltpu.sync_copy(cnt_sh.at[pl.ds(off, zs)], rows0.at[pl.ds(0, zs)])
            pltpu.sync_copy(rows0.at[pl.ds(0, zs)],
                            cnt_hbm.at[pl.ds(cid * _NPAD + off, zs)])

    xs, cnt = gk(x, src, dst, consts)
    return xs.reshape(e, d), cnt.reshape(_NC, _NPAD, d)[:, :n]


def _sc_scatter(msg, dst, n, e, d):
    nch = e // _K
    msg3 = msg.reshape(nch, _K, d)
    zeros = jnp.zeros((_K, d), jnp.float32)

    @functools.partial(
        pl.kernel,
        out_type=jax.ShapeDtypeStruct((_NC * _NPAD, d), jnp.float32),
        mesh=_sc_mesh(),
        scratch_types=[
            pltpu.VMEM((_K,), jnp.int32),
            pltpu.VMEM((_K,), jnp.int32),
            pltpu.VMEM((_K, d), jnp.float32),
            pltpu.VMEM((_K, d), jnp.float32),
            pltpu.VMEM_SHARED((_NPAD, d), jnp.float32),
            pltpu.SemaphoreType.DMA,
            pltpu.SemaphoreType.DMA,
            pltpu.SemaphoreType.DMA,
            pltpu.SemaphoreType.DMA,
        ],
    )
    def sk(msg_hbm, dst_hbm, zero_hbm, out_hbm, idxd0, idxd1, buf0, buf1,
           acc_sh, sem0, sem1, dsem0, dsem1):
        cid = lax.axis_index("c")
        sid = lax.axis_index("s")
        wid = cid * _NS + sid
        s, nj, start, sh = _worker_range(wid, nch)

        pltpu.sync_copy(zero_hbm, buf0)

        for zo, zs in _ZCHUNKS:
            pltpu.sync_copy(buf0.at[pl.ds(0, zs)],
                            acc_sh.at[pl.ds(sid * _NT + zo, zs)])
        plsc.subcore_barrier()

        idxd = (idxd0, idxd1)
        bufs = (buf0, buf1)
        sems = (sem0, sem1)
        dsems = (dsem0, dsem1)

        def msg_start(j, b):
            pltpu.async_copy(msg_hbm.at[s + j], bufs[b], sems[b])
            pltpu.async_copy(dst_hbm.at[pl.ds((s + j) * _K, _K)], idxd[b],
                             dsems[b])

        def msg_wait(j, b):
            pltpu.make_async_copy(msg_hbm.at[s + j], bufs[b],
                                  sems[b]).wait()
            pltpu.make_async_copy(dst_hbm.at[pl.ds((s + j) * _K, _K)],
                                  idxd[b], dsems[b]).wait()

        @pl.when(nj > 0)
        def _():
            msg_start(0, 0)

        def pair(g, carry):
            for b in (0, 1):
                j = 2 * g + b

                @pl.when(j < nj)
                def _():
                    @pl.when(j + 1 < nj)
                    def _():
                        msg_start(j + 1, 1 - b)

                    msg_wait(j, b)
                    pltpu.sync_copy(bufs[b], acc_sh.at[idxd[b]], add=True)
            return carry

        lax.fori_loop(0, (nch // _NW + 2) // 2, pair, 0)
        plsc.subcore_barrier()

        for zo, zs in _ZCHUNKS:
            off = sid * _NT + zo
            pltpu.sync_copy(acc_sh.at[pl.ds(off, zs)], buf0.at[pl.ds(0, zs)])
            pltpu.sync_copy(buf0.at[pl.ds(0, zs)],
                            out_hbm.at[pl.ds(cid * _NPAD + off, zs)])

    return sk(msg3, dst, zeros).reshape(_NC, _NPAD, d)[:, :n]


def _edge_mlp(xs, ea, w1a, w1b, b1, w2, b2, e, d, h):
    be = 4000
    grid = e // be

    bf = jnp.bfloat16

    def body(xs_ref, ea_ref, w1a_ref, w1b_ref, b1_ref, w2_ref, b2_ref, o_ref):
        cat = jnp.concatenate(
            [xs_ref[...].astype(bf), ea_ref[...].astype(bf)], axis=1)
        w1 = jnp.concatenate([w1a_ref[...], w1b_ref[...]], axis=0)
        hv = jnp.dot(cat, w1, preferred_element_type=jnp.float32) + b1_ref[...]
        hv = jnp.maximum(hv, 0.0)
        o_ref[...] = (jnp.dot(hv.astype(bf), w2_ref[...],
                              preferred_element_type=jnp.float32)
                      + b2_ref[...])

    return pl.pallas_call(
        body,
        grid=(grid,),
        in_specs=[
            pl.BlockSpec((be, d), lambda i: (i, 0)),
            pl.BlockSpec((be, d), lambda i: (i, 0)),
            pl.BlockSpec((d, h), lambda i: (0, 0)),
            pl.BlockSpec((d, h), lambda i: (0, 0)),
            pl.BlockSpec((1, h), lambda i: (0, 0)),
            pl.BlockSpec((h, d), lambda i: (0, 0)),
            pl.BlockSpec((1, d), lambda i: (0, 0)),
        ],
        out_specs=pl.BlockSpec((be, d), lambda i: (i, 0)),
        out_shape=jax.ShapeDtypeStruct((e, d), jnp.float32),
    )(xs, ea, w1a, w1b, b1, w2, b2)


def _node_mlp(x, p0, p1, c0, c1, u, batch3, w1a, w1b, w1c, b1, w2, b2,
              n, d, nb, gd, h):
    bn = 1000
    grid = n // bn

    def body(x_ref, p0_ref, p1_ref, c0_ref, c1_ref, u_ref, b_ref,
             w1a_ref, w1b_ref, w1c_ref, b1_ref, w2_ref, b2_ref,
             y_ref, s0_ref, s1_ref, s2_ref):
        i = pl.program_id(0)
        xv = x_ref[...]
        p = p0_ref[...] + p1_ref[...]
        cnt = jnp.max(c0_ref[...] + c1_ref[...], axis=1, keepdims=True)
        agg = p / jnp.clip(cnt, 1.0, None)
        bv = b_ref[0, 0, :]
        oh = (bv[:, None] == lax.broadcasted_iota(jnp.int32, (bn, nb), 1)
              ).astype(jnp.float32)
        ub = jnp.dot(oh, u_ref[...], preferred_element_type=jnp.float32)
        hv = (jnp.dot(xv, w1a_ref[...], preferred_element_type=jnp.float32)
              + jnp.dot(agg, w1b_ref[...], preferred_element_type=jnp.float32)
              + jnp.dot(ub, w1c_ref[...], preferred_element_type=jnp.float32)
              + b1_ref[...])
        hv = jnp.maximum(hv, 0.0)
        y = (jnp.dot(hv, w2_ref[...], preferred_element_type=jnp.float32)
             + b2_ref[...] + xv)
        y_ref[...] = y

        dg = lambda a, b: lax.dot_general(
            a, b, (((0,), (0,)), ((), ())), preferred_element_type=jnp.float32)

        @pl.when(i == 0)
        def _():
            s0_ref[...] = jnp.zeros_like(s0_ref)
            s1_ref[...] = jnp.zeros_like(s1_ref)
            s2_ref[...] = jnp.zeros_like(s2_ref)

        s0_ref[...] += dg(oh, jnp.ones_like(y))
        s1_ref[...] += dg(oh, y)
        s2_ref[...] += dg(oh, y * y)

    return pl.pallas_call(
        body,
        grid=(grid,),
        in_specs=[
            pl.BlockSpec((bn, d), lambda i: (i, 0)),
            pl.BlockSpec((bn, d), lambda i: (i, 0)),
            pl.BlockSpec((bn, d), lambda i: (i, 0)),
            pl.BlockSpec((bn, d), lambda i: (i, 0)),
            pl.BlockSpec((bn, d), lambda i: (i, 0)),
            pl.BlockSpec((nb, gd), lambda i: (0, 0)),
            pl.BlockSpec((1, 1, bn), lambda i: (i, 0, 0)),
            pl.BlockSpec((d, h), lambda i: (0, 0)),
            pl.BlockSpec((d, h), lambda i: (0, 0)),
            pl.BlockSpec((gd, h), lambda i: (0, 0)),
            pl.BlockSpec((1, h), lambda i: (0, 0)),
            pl.BlockSpec((h, d), lambda i: (0, 0)),
            pl.BlockSpec((1, d), lambda i: (0, 0)),
        ],
        out_specs=[
            pl.BlockSpec((bn, d), lambda i: (i, 0)),
            pl.BlockSpec((nb, d), lambda i: (0, 0)),
            pl.BlockSpec((nb, d), lambda i: (0, 0)),
            pl.BlockSpec((nb, d), lambda i: (0, 0)),
        ],
        out_shape=[
            jax.ShapeDtypeStruct((n, d), jnp.float32),
            jax.ShapeDtypeStruct((nb, d), jnp.float32),
            jax.ShapeDtypeStruct((nb, d), jnp.float32),
            jax.ShapeDtypeStruct((nb, d), jnp.float32),
        ],
    )(x, p0, p1, c0, c1, u, batch3, w1a, w1b, w1c, b1, w2, b2)


def _graphnorm(y, batch3, s0, s1, s2, alpha, gamma, beta, n, d, nb):
    bn = 1000
    grid = n // bn

    def body(y_ref, b_ref, s0_ref, s1_ref, s2_ref, al_ref, ga_ref, be_ref,
             o_ref):
        gcnt = jnp.clip(s0_ref[...], 1.0, None)
        mean = s1_ref[...] / gcnt
        am = al_ref[...] * mean
        var = s2_ref[...] / gcnt - am * (2.0 * mean - am)
        scale = ga_ref[...] / jnp.sqrt(var + 1e-5)
        shift = be_ref[...] - scale * am
        bv = b_ref[0, 0, :]
        oh = (bv[:, None] == lax.broadcasted_iota(jnp.int32, (bn, nb), 1)
              ).astype(jnp.float32)
        o_ref[...] = (jnp.dot(oh, scale, preferred_element_type=jnp.float32)
                      * y_ref[...]
                      + jnp.dot(oh, shift, preferred_element_type=jnp.float32))

    return pl.pallas_call(
        body,
        grid=(grid,),
        in_specs=[
            pl.BlockSpec((bn, d), lambda i: (i, 0)),
            pl.BlockSpec((1, 1, bn), lambda i: (i, 0, 0)),
            pl.BlockSpec((nb, d), lambda i: (0, 0)),
            pl.BlockSpec((nb, d), lambda i: (0, 0)),
            pl.BlockSpec((nb, d), lambda i: (0, 0)),
            pl.BlockSpec((1, d), lambda i: (0, 0)),
            pl.BlockSpec((1, d), lambda i: (0, 0)),
            pl.BlockSpec((1, d), lambda i: (0, 0)),
        ],
        out_specs=pl.BlockSpec((bn, d), lambda i: (i, 0)),
        out_shape=jax.ShapeDtypeStruct((n, d), jnp.float32),
    )(y, batch3, s0, s1, s2, alpha, gamma, beta)


def kernel(x, edge_index, edge_attr, u, batch, agg_w1, agg_b1, agg_w2, agg_b2,
           upd_w1, upd_b1, upd_w2, upd_b2, gn_alpha, gn_gamma, gn_beta):
    n, d = x.shape
    e = edge_index.shape[1]
    nb, gd = u.shape
    h = agg_w1.shape[1]
    hu = upd_w1.shape[1]

    src = edge_index[0]
    dst = edge_index[1]

    xs, cnts = _sc_gather_count(x, src, dst, n, e, d)

    bf = jnp.bfloat16
    msg = _edge_mlp(xs, edge_attr, agg_w1[:d].astype(bf), agg_w1[d:].astype(bf),
                    agg_b1.reshape(1, h), agg_w2.astype(bf),
                    agg_b2.reshape(1, d), e, d, h)

    parts = _sc_scatter(msg, dst, n, e, d)

    batch3 = batch.reshape(n // 1000, 1, 1000)
    y, s0, s1, s2 = _node_mlp(
        x, parts[0], parts[1], cnts[0], cnts[1], u, batch3,
        upd_w1[:d], upd_w1[d:2 * d], upd_w1[2 * d:],
        upd_b1.reshape(1, hu), upd_w2, upd_b2.reshape(1, d),
        n, d, nb, gd, hu)

    return _graphnorm(y, batch3, s0, s1, s2,
                      gn_alpha.reshape(1, d), gn_gamma.reshape(1, d),
                      gn_beta.reshape(1, d), n, d, nb)


# speedup vs baseline: 4.3779x; 1.0484x over previous
import functools

import jax
import jax.numpy as jnp
from jax import lax
from jax.experimental import pallas as pl
from jax.experimental.pallas import tpu as pltpu
from jax.experimental.pallas import tpu_sc as plsc

_NC = 2
_NS = 16
_NW = _NC * _NS
_K = 128
_NT = 632
_NPAD = _NT * _NS
_ZCHUNKS = ((0, 128), (128, 128), (256, 128), (384, 128), (512, 120))


_MAXJ = 79


def _sc_mesh():
    return plsc.VectorSubcoreMesh(
        core_axis_name="c", subcore_axis_name="s",
        num_cores=_NC, num_subcores=_NS)


def _worker_range(wid, nch):
    base = nch // _NW
    ext = nch % _NW
    s = base * wid + jnp.minimum(wid, ext)
    nj = base + jnp.where(wid < ext, 1, 0)
    start = jnp.minimum(s, nch - _MAXJ)
    sh = s - start
    return s, nj, start, sh


def _sc_gather_count(x, src, dst, n, e, d):
    nch = e // _K
    consts = jnp.stack([jnp.zeros((_K, d), jnp.float32),
                        jnp.ones((_K, d), jnp.float32)])

    @functools.partial(
        pl.kernel,
        out_type=(jax.ShapeDtypeStruct((nch, _K, d), jnp.float32),
                  jax.ShapeDtypeStruct((_NC * _NPAD, d), jnp.float32)),
        mesh=_sc_mesh(),
        scratch_types=[
            pltpu.VMEM((_K,), jnp.int32),
            pltpu.VMEM((_K,), jnp.int32),
            pltpu.VMEM((_K,), jnp.int32),
            pltpu.VMEM((_K,), jnp.int32),
            pltpu.VMEM((_K, d), jnp.float32),
            pltpu.VMEM((_K, d), jnp.float32),
            pltpu.VMEM((_K, d), jnp.float32),
            pltpu.VMEM_SHARED((_NPAD, d), jnp.float32),
            pltpu.SemaphoreType.DMA,
            pltpu.SemaphoreType.DMA,
            pltpu.SemaphoreType.DMA,
            pltpu.SemaphoreType.DMA,
            pltpu.SemaphoreType.DMA,
            pltpu.SemaphoreType.DMA,
        ],
    )
    def gk(x_hbm, src_hbm, dst_hbm, const_hbm, out_hbm, cnt_hbm,
           idxs0, idxs1, idxd0, idxd1, rows0, rows1, ones_v, cnt_sh,
           sem0, sem1, dsem0, dsem1, isem0, isem1):
        cid = lax.axis_index("c")
        sid = lax.axis_index("s")
        wid = cid * _NS + sid
        s, nj, start, sh = _worker_range(wid, nch)

        pltpu.sync_copy(const_hbm.at[0], rows0)
        pltpu.sync_copy(const_hbm.at[1], ones_v)

        for zo, zs in _ZCHUNKS:
            pltpu.sync_copy(rows0.at[pl.ds(0, zs)],
                            cnt_sh.at[pl.ds(sid * _NT + zo, zs)])
        plsc.subcore_barrier()

        idxs = (idxs0, idxs1)
        idxd = (idxd0, idxd1)
        rows = (rows0, rows1)
        sems = (sem0, sem1)
        dsems = (dsem0, dsem1)
        isems = (isem0, isem1)

        def sidx_start(j, b):
            pltpu.async_copy(src_hbm.at[pl.ds((s + j) * _K, _K)], idxs[b],
                             isems[b])

        def sidx_wait(j, b):
            pltpu.make_async_copy(src_hbm.at[pl.ds((s + j) * _K, _K)],
                                  idxs[b], isems[b]).wait()

        def gather_start(j, b):
            pltpu.async_copy(x_hbm.at[idxs[b]], rows[b], sems[b])

        def gather_wait(j, b):
            pltpu.make_async_copy(x_hbm.at[idxs[b]], rows[b], sems[b]).wait()

        def didx_start(j, b):
            pltpu.async_copy(dst_hbm.at[pl.ds((s + j) * _K, _K)], idxd[b],
                             dsems[b])

        def didx_wait(j, b):
            pltpu.make_async_copy(dst_hbm.at[pl.ds((s + j) * _K, _K)],
                                  idxd[b], dsems[b]).wait()

        @pl.when(nj > 0)
        def _():
            pltpu.sync_copy(src_hbm.at[pl.ds(s * _K, _K)], idxs0)
            gather_start(0, 0)
            didx_start(0, 0)

        @pl.when(nj > 1)
        def _():
            sidx_start(1, 1)

        def pair(g, carry):
            for b in (0, 1):
                j = 2 * g + b

                @pl.when(j < nj)
                def _():
                    @pl.when(j + 1 < nj)
                    def _():
                        sidx_wait(j + 1, 1 - b)
                        gather_start(j + 1, 1 - b)
                        didx_start(j + 1, 1 - b)

                    gather_wait(j, b)
                    pltpu.sync_copy(rows[b], out_hbm.at[s + j])
                    didx_wait(j, b)
                    pltpu.sync_copy(ones_v, cnt_sh.at[idxd[b]], add=True)

                    @pl.when(j + 2 < nj)
                    def _():
                        sidx_start(j + 2, b)
            return carry

        lax.fori_loop(0, (nch // _NW + 2) // 2, pair, 0)
        plsc.subcore_barrier()

        for zo, zs in _ZCHUNKS:
            off = sid * _NT + zo
            pltpu.sync_copy(cnt_sh.at[pl.ds(off, zs)], rows0.at[pl.ds(0, zs)])
            pltpu.sync_copy(rows0.at[pl.ds(0, zs)],
                            cnt_hbm.at[pl.ds(cid * _NPAD + off, zs)])

    xs, cnt = gk(x, src, dst, consts)
    return xs.reshape(e, d), cnt.reshape(_NC, _NPAD, d)[:, :n]


def _sc_scatter(msg, dst, n, e, d):
    nch = e // _K
    msg3 = msg.reshape(nch, _K, d)
    zeros = jnp.zeros((_K, d), jnp.float32)

    @functools.partial(
        pl.kernel,
        out_type=jax.ShapeDtypeStruct((_NC * _NPAD, d), jnp.float32),
        mesh=_sc_mesh(),
        scratch_types=[
            pltpu.VMEM((_K,), jnp.int32),
            pltpu.VMEM((_K,), jnp.int32),
            pltpu.VMEM((_K, d), jnp.float32),
            pltpu.VMEM((_K, d), jnp.float32),
            pltpu.VMEM_SHARED((_NPAD, d), jnp.float32),
            pltpu.SemaphoreType.DMA,
            pltpu.SemaphoreType.DMA,
            pltpu.SemaphoreType.DMA,
            pltpu.SemaphoreType.DMA,
        ],
    )
    def sk(msg_hbm, dst_hbm, zero_hbm, out_hbm, idxd0, idxd1, buf0, buf1,
           acc_sh, sem0, sem1, dsem0, dsem1):
        cid = lax.axis_index("c")
        sid = lax.axis_index("s")
        wid = cid * _NS + sid
        s, nj, start, sh = _worker_range(wid, nch)

        pltpu.sync_copy(zero_hbm, buf0)

        for zo, zs in _ZCHUNKS:
            pltpu.sync_copy(buf0.at[pl.ds(0, zs)],
                            acc_sh.at[pl.ds(sid * _NT + zo, zs)])
        plsc.subcore_barrier()

        idxd = (idxd0, idxd1)
        bufs = (buf0, buf1)
        sems = (sem0, sem1)
        dsems = (dsem0, dsem1)

        def msg_start(j, b):
            pltpu.async_copy(msg_hbm.at[s + j], bufs[b], sems[b])
            pltpu.async_copy(dst_hbm.at[pl.ds((s + j) * _K, _K)], idxd[b],
                             dsems[b])

        def msg_wait(j, b):
            pltpu.make_async_copy(msg_hbm.at[s + j], bufs[b],
                                  sems[b]).wait()
            pltpu.make_async_copy(dst_hbm.at[pl.ds((s + j) * _K, _K)],
                                  idxd[b], dsems[b]).wait()

        @pl.when(nj > 0)
        def _():
            msg_start(0, 0)

        def pair(g, carry):
            for b in (0, 1):
                j = 2 * g + b

                @pl.when(j < nj)
                def _():
                    @pl.when(j + 1 < nj)
                    def _():
                        msg_start(j + 1, 1 - b)

                    msg_wait(j, b)
                    pltpu.sync_copy(bufs[b], acc_sh.at[idxd[b]], add=True)
            return carry

        lax.fori_loop(0, (nch // _NW + 2) // 2, pair, 0)
        plsc.subcore_barrier()

        for zo, zs in _ZCHUNKS:
            off = sid * _NT + zo
            pltpu.sync_copy(acc_sh.at[pl.ds(off, zs)], buf0.at[pl.ds(0, zs)])
            pltpu.sync_copy(buf0.at[pl.ds(0, zs)],
                            out_hbm.at[pl.ds(cid * _NPAD + off, zs)])

    return sk(msg3, dst, zeros).reshape(_NC, _NPAD, d)[:, :n]


def _edge_mlp(xs, ea, w1a, w1b, b1, w2, b2, e, d, h):
    be = 8000
    grid = e // be

    bf = jnp.bfloat16

    def body(xs_ref, ea_ref, w1a_ref, w1b_ref, b1_ref, w2_ref, b2_ref, o_ref):
        cat = jnp.concatenate(
            [xs_ref[...].astype(bf), ea_ref[...].astype(bf)], axis=1)
        w1 = jnp.concatenate([w1a_ref[...], w1b_ref[...]], axis=0)
        hv = jnp.dot(cat, w1, preferred_element_type=jnp.float32) + b1_ref[...]
        hv = jnp.maximum(hv, 0.0)
        o_ref[...] = (jnp.dot(hv.astype(bf), w2_ref[...],
                              preferred_element_type=jnp.float32)
                      + b2_ref[...])

    return pl.pallas_call(
        body,
        grid=(grid,),
        in_specs=[
            pl.BlockSpec((be, d), lambda i: (i, 0)),
            pl.BlockSpec((be, d), lambda i: (i, 0)),
            pl.BlockSpec((d, h), lambda i: (0, 0)),
            pl.BlockSpec((d, h), lambda i: (0, 0)),
            pl.BlockSpec((1, h), lambda i: (0, 0)),
            pl.BlockSpec((h, d), lambda i: (0, 0)),
            pl.BlockSpec((1, d), lambda i: (0, 0)),
        ],
        out_specs=pl.BlockSpec((be, d), lambda i: (i, 0)),
        out_shape=jax.ShapeDtypeStruct((e, d), jnp.float32),
    )(xs, ea, w1a, w1b, b1, w2, b2)


def _node_mlp(x, p0, p1, c0, c1, u, batch3, w1a, w1b, w1c, b1, w2, b2,
              n, d, nb, gd, h):
    bn = 1000
    grid = n // bn

    def body(x_ref, p0_ref, p1_ref, c0_ref, c1_ref, u_ref, b_ref,
             w1a_ref, w1b_ref, w1c_ref, b1_ref, w2_ref, b2_ref,
             y_ref, s0_ref, s1_ref, s2_ref):
        i = pl.program_id(0)
        xv = x_ref[...]
        p = p0_ref[...] + p1_ref[...]
        cnt = jnp.max(c0_ref[...] + c1_ref[...], axis=1, keepdims=True)
        agg = p / jnp.clip(cnt, 1.0, None)
        bv = b_ref[0, 0, :]
        oh = (bv[:, None] == lax.broadcasted_iota(jnp.int32, (bn, nb), 1)
              ).astype(jnp.float32)
        ub = jnp.dot(oh, u_ref[...], preferred_element_type=jnp.float32)
        hv = (jnp.dot(xv, w1a_ref[...], preferred_element_type=jnp.float32)
              + jnp.dot(agg, w1b_ref[...], preferred_element_type=jnp.float32)
              + jnp.dot(ub, w1c_ref[...], preferred_element_type=jnp.float32)
              + b1_ref[...])
        hv = jnp.maximum(hv, 0.0)
        y = (jnp.dot(hv, w2_ref[...], preferred_element_type=jnp.float32)
             + b2_ref[...] + xv)
        y_ref[...] = y

        dg = lambda a, b: lax.dot_general(
            a, b, (((0,), (0,)), ((), ())), preferred_element_type=jnp.float32)

        @pl.when(i == 0)
        def _():
            s0_ref[...] = jnp.zeros_like(s0_ref)
            s1_ref[...] = jnp.zeros_like(s1_ref)
            s2_ref[...] = jnp.zeros_like(s2_ref)

        s0_ref[...] += dg(oh, jnp.ones_like(y))
        s1_ref[...] += dg(oh, y)
        s2_ref[...] += dg(oh, y * y)

    return pl.pallas_call(
        body,
        grid=(grid,),
        in_specs=[
            pl.BlockSpec((bn, d), lambda i: (i, 0)),
            pl.BlockSpec((bn, d), lambda i: (i, 0)),
            pl.BlockSpec((bn, d), lambda i: (i, 0)),
            pl.BlockSpec((bn, d), lambda i: (i, 0)),
            pl.BlockSpec((bn, d), lambda i: (i, 0)),
            pl.BlockSpec((nb, gd), lambda i: (0, 0)),
            pl.BlockSpec((1, 1, bn), lambda i: (i, 0, 0)),
            pl.BlockSpec((d, h), lambda i: (0, 0)),
            pl.BlockSpec((d, h), lambda i: (0, 0)),
            pl.BlockSpec((gd, h), lambda i: (0, 0)),
            pl.BlockSpec((1, h), lambda i: (0, 0)),
            pl.BlockSpec((h, d), lambda i: (0, 0)),
            pl.BlockSpec((1, d), lambda i: (0, 0)),
        ],
        out_specs=[
            pl.BlockSpec((bn, d), lambda i: (i, 0)),
            pl.BlockSpec((nb, d), lambda i: (0, 0)),
            pl.BlockSpec((nb, d), lambda i: (0, 0)),
            pl.BlockSpec((nb, d), lambda i: (0, 0)),
        ],
        out_shape=[
            jax.ShapeDtypeStruct((n, d), jnp.float32),
            jax.ShapeDtypeStruct((nb, d), jnp.float32),
            jax.ShapeDtypeStruct((nb, d), jnp.float32),
            jax.ShapeDtypeStruct((nb, d), jnp.float32),
        ],
    )(x, p0, p1, c0, c1, u, batch3, w1a, w1b, w1c, b1, w2, b2)


def _graphnorm(y, batch3, s0, s1, s2, alpha, gamma, beta, n, d, nb):
    bn = 1000
    grid = n // bn

    def body(y_ref, b_ref, s0_ref, s1_ref, s2_ref, al_ref, ga_ref, be_ref,
             o_ref):
        gcnt = jnp.clip(s0_ref[...], 1.0, None)
        mean = s1_ref[...] / gcnt
        am = al_ref[...] * mean
        var = s2_ref[...] / gcnt - am * (2.0 * mean - am)
        scale = ga_ref[...] / jnp.sqrt(var + 1e-5)
        shift = be_ref[...] - scale * am
        bv = b_ref[0, 0, :]
        oh = (bv[:, None] == lax.broadcasted_iota(jnp.int32, (bn, nb), 1)
              ).astype(jnp.float32)
        o_ref[...] = (jnp.dot(oh, scale, preferred_element_type=jnp.float32)
                      * y_ref[...]
                      + jnp.dot(oh, shift, preferred_element_type=jnp.float32))

    return pl.pallas_call(
        body,
        grid=(grid,),
        in_specs=[
            pl.BlockSpec((bn, d), lambda i: (i, 0)),
            pl.BlockSpec((1, 1, bn), lambda i: (i, 0, 0)),
            pl.BlockSpec((nb, d), lambda i: (0, 0)),
            pl.BlockSpec((nb, d), lambda i: (0, 0)),
            pl.BlockSpec((nb, d), lambda i: (0, 0)),
            pl.BlockSpec((1, d), lambda i: (0, 0)),
            pl.BlockSpec((1, d), lambda i: (0, 0)),
            pl.BlockSpec((1, d), lambda i: (0, 0)),
        ],
        out_specs=pl.BlockSpec((bn, d), lambda i: (i, 0)),
        out_shape=jax.ShapeDtypeStruct((n, d), jnp.float32),
    )(y, batch3, s0, s1, s2, alpha, gamma, beta)


def kernel(x, edge_index, edge_attr, u, batch, agg_w1, agg_b1, agg_w2, agg_b2,
           upd_w1, upd_b1, upd_w2, upd_b2, gn_alpha, gn_gamma, gn_beta):
    n, d = x.shape
    e = edge_index.shape[1]
    nb, gd = u.shape
    h = agg_w1.shape[1]
    hu = upd_w1.shape[1]

    src = edge_index[0]
    dst = edge_index[1]

    xs, cnts = _sc_gather_count(x, src, dst, n, e, d)

    bf = jnp.bfloat16
    msg = _edge_mlp(xs, edge_attr, agg_w1[:d].astype(bf), agg_w1[d:].astype(bf),
                    agg_b1.reshape(1, h), agg_w2.astype(bf),
                    agg_b2.reshape(1, d), e, d, h)

    parts = _sc_scatter(msg, dst, n, e, d)

    batch3 = batch.reshape(n // 1000, 1, 1000)
    y, s0, s1, s2 = _node_mlp(
        x, parts[0], parts[1], cnts[0], cnts[1], u, batch3,
        upd_w1[:d], upd_w1[d:2 * d], upd_w1[2 * d:],
        upd_b1.reshape(1, hu), upd_w2, upd_b2.reshape(1, d),
        n, d, nb, gd, hu)

    return _graphnorm(y, batch3, s0, s1, s2,
                      gn_alpha.reshape(1, d), gn_gamma.reshape(1, d),
                      gn_beta.reshape(1, d), n, d, nb)


# speedup vs baseline: 4.5007x; 1.0280x over previous
import functools

import jax
import jax.numpy as jnp
from jax import lax
from jax.experimental import pallas as pl
from jax.experimental.pallas import tpu as pltpu
from jax.experimental.pallas import tpu_sc as plsc

_NC = 2
_NS = 16
_NW = _NC * _NS
_K = 128
_NT = 632
_NPAD = _NT * _NS
_ZCHUNKS = ((0, 128), (128, 128), (256, 128), (384, 128), (512, 120))


_MAXJ = 79


def _sc_mesh():
    return plsc.VectorSubcoreMesh(
        core_axis_name="c", subcore_axis_name="s",
        num_cores=_NC, num_subcores=_NS)


def _worker_range(wid, nch):
    base = nch // _NW
    ext = nch % _NW
    s = base * wid + jnp.minimum(wid, ext)
    nj = base + jnp.where(wid < ext, 1, 0)
    start = jnp.minimum(s, nch - _MAXJ)
    sh = s - start
    return s, nj, start, sh


def _sc_gather_count(x, src, dst, n, e, d):
    nch = e // _K
    consts = jnp.stack([jnp.zeros((_K, d), jnp.float32),
                        jnp.ones((_K, d), jnp.float32)])

    @functools.partial(
        pl.kernel,
        out_type=(jax.ShapeDtypeStruct((nch, _K, d), jnp.float32),
                  jax.ShapeDtypeStruct((_NC * _NPAD, d), jnp.float32)),
        mesh=_sc_mesh(),
        scratch_types=[
            pltpu.VMEM((_K,), jnp.int32),
            pltpu.VMEM((_K,), jnp.int32),
            pltpu.VMEM((_K,), jnp.int32),
            pltpu.VMEM((_K,), jnp.int32),
            pltpu.VMEM((_K, d), jnp.float32),
            pltpu.VMEM((_K, d), jnp.float32),
            pltpu.VMEM((_K, d), jnp.float32),
            pltpu.VMEM_SHARED((_NPAD, d), jnp.float32),
            pltpu.SemaphoreType.DMA,
            pltpu.SemaphoreType.DMA,
            pltpu.SemaphoreType.DMA,
            pltpu.SemaphoreType.DMA,
            pltpu.SemaphoreType.DMA,
            pltpu.SemaphoreType.DMA,
            pltpu.SemaphoreType.DMA,
            pltpu.SemaphoreType.DMA,
        ],
    )
    def gk(x_hbm, src_hbm, dst_hbm, const_hbm, out_hbm, cnt_hbm,
           idxs0, idxs1, idxd0, idxd1, rows0, rows1, ones_v, cnt_sh,
           sem0, sem1, dsem0, dsem1, isem0, isem1, wsem0, wsem1):
        cid = lax.axis_index("c")
        sid = lax.axis_index("s")
        wid = cid * _NS + sid
        s, nj, start, sh = _worker_range(wid, nch)

        pltpu.sync_copy(const_hbm.at[0], rows0)
        pltpu.sync_copy(const_hbm.at[1], ones_v)

        for zo, zs in _ZCHUNKS:
            pltpu.sync_copy(rows0.at[pl.ds(0, zs)],
                            cnt_sh.at[pl.ds(sid * _NT + zo, zs)])
        plsc.subcore_barrier()

        idxs = (idxs0, idxs1)
        idxd = (idxd0, idxd1)
        rows = (rows0, rows1)
        sems = (sem0, sem1)
        dsems = (dsem0, dsem1)
        isems = (isem0, isem1)
        wsems = (wsem0, wsem1)

        def write_start(j, b):
            pltpu.async_copy(rows[b], out_hbm.at[s + j], wsems[b])

        def write_wait(j, b):
            pltpu.make_async_copy(rows[b], out_hbm.at[s + j], wsems[b]).wait()

        def sidx_start(j, b):
            pltpu.async_copy(src_hbm.at[pl.ds((s + j) * _K, _K)], idxs[b],
                             isems[b])

        def sidx_wait(j, b):
            pltpu.make_async_copy(src_hbm.at[pl.ds((s + j) * _K, _K)],
                                  idxs[b], isems[b]).wait()

        def gather_start(j, b):
            pltpu.async_copy(x_hbm.at[idxs[b]], rows[b], sems[b])

        def gather_wait(j, b):
            pltpu.make_async_copy(x_hbm.at[idxs[b]], rows[b], sems[b]).wait()

        def didx_start(j, b):
            pltpu.async_copy(dst_hbm.at[pl.ds((s + j) * _K, _K)], idxd[b],
                             dsems[b])

        def didx_wait(j, b):
            pltpu.make_async_copy(dst_hbm.at[pl.ds((s + j) * _K, _K)],
                                  idxd[b], dsems[b]).wait()

        @pl.when(nj > 0)
        def _():
            pltpu.sync_copy(src_hbm.at[pl.ds(s * _K, _K)], idxs0)
            gather_start(0, 0)
            didx_start(0, 0)

        @pl.when(nj > 1)
        def _():
            sidx_start(1, 1)

        def pair(g, carry):
            for b in (0, 1):
                j = 2 * g + b

                @pl.when(j < nj)
                def _():
                    @pl.when(j + 1 < nj)
                    def _():
                        sidx_wait(j + 1, 1 - b)

                        @pl.when(j >= 1)
                        def _():
                            write_wait(j - 1, 1 - b)

                        gather_start(j + 1, 1 - b)
                        didx_start(j + 1, 1 - b)

                    gather_wait(j, b)
                    write_start(j, b)
                    didx_wait(j, b)
                    pltpu.sync_copy(ones_v, cnt_sh.at[idxd[b]], add=True)

                    @pl.when(j + 2 < nj)
                    def _():
                        sidx_start(j + 2, b)
            return carry

        lax.fori_loop(0, (nch // _NW + 2) // 2, pair, 0)
        for b in (0, 1):
            pltpu.make_async_copy(rows[b], out_hbm.at[s], wsems[b]).wait()
        plsc.subcore_barrier()

        for zo, zs in _ZCHUNKS:
            off = sid * _NT + zo
            pltpu.sync_copy(cnt_sh.at[pl.ds(off, zs)], rows0.at[pl.ds(0, zs)])
            pltpu.sync_copy(rows0.at[pl.ds(0, zs)],
                            cnt_hbm.at[pl.ds(cid * _NPAD + off, zs)])

    xs, cnt = gk(x, src, dst, consts)
    return xs.reshape(e, d), cnt.reshape(_NC, _NPAD, d)


def _sc_scatter(msg, dst, n, e, d):
    nch = e // _K
    msg3 = msg.reshape(nch, _K, d)
    zeros = jnp.zeros((_K, d), jnp.float32)

    @functools.partial(
        pl.kernel,
        out_type=jax.ShapeDtypeStruct((_NC * _NPAD, d), jnp.float32),
        mesh=_sc_mesh(),
        scratch_types=[
            pltpu.VMEM((_K,), jnp.int32),
            pltpu.VMEM((_K,), jnp.int32),
            pltpu.VMEM((_K, d), jnp.float32),
            pltpu.VMEM((_K, d), jnp.float32),
            pltpu.VMEM_SHARED((_NPAD, d), jnp.float32),
            pltpu.SemaphoreType.DMA,
            pltpu.SemaphoreType.DMA,
            pltpu.SemaphoreType.DMA,
            pltpu.SemaphoreType.DMA,
        ],
    )
    def sk(msg_hbm, dst_hbm, zero_hbm, out_hbm, idxd0, idxd1, buf0, buf1,
           acc_sh, sem0, sem1, dsem0, dsem1):
        cid = lax.axis_index("c")
        sid = lax.axis_index("s")
        wid = cid * _NS + sid
        s, nj, start, sh = _worker_range(wid, nch)

        pltpu.sync_copy(zero_hbm, buf0)

        for zo, zs in _ZCHUNKS:
            pltpu.sync_copy(buf0.at[pl.ds(0, zs)],
                            acc_sh.at[pl.ds(sid * _NT + zo, zs)])
        plsc.subcore_barrier()

        idxd = (idxd0, idxd1)
        bufs = (buf0, buf1)
        sems = (sem0, sem1)
        dsems = (dsem0, dsem1)

        def msg_start(j, b):
            pltpu.async_copy(msg_hbm.at[s + j], bufs[b], sems[b])
            pltpu.async_copy(dst_hbm.at[pl.ds((s + j) * _K, _K)], idxd[b],
                             dsems[b])

        def msg_wait(j, b):
            pltpu.make_async_copy(msg_hbm.at[s + j], bufs[b],
                                  sems[b]).wait()
            pltpu.make_async_copy(dst_hbm.at[pl.ds((s + j) * _K, _K)],
                                  idxd[b], dsems[b]).wait()

        @pl.when(nj > 0)
        def _():
            msg_start(0, 0)

        def pair(g, carry):
            for b in (0, 1):
                j = 2 * g + b

                @pl.when(j < nj)
                def _():
                    @pl.when(j + 1 < nj)
                    def _():
                        msg_start(j + 1, 1 - b)

                    msg_wait(j, b)
                    pltpu.sync_copy(bufs[b], acc_sh.at[idxd[b]], add=True)
            return carry

        lax.fori_loop(0, (nch // _NW + 2) // 2, pair, 0)
        plsc.subcore_barrier()

        for zo, zs in _ZCHUNKS:
            off = sid * _NT + zo
            pltpu.sync_copy(acc_sh.at[pl.ds(off, zs)], buf0.at[pl.ds(0, zs)])
            pltpu.sync_copy(buf0.at[pl.ds(0, zs)],
                            out_hbm.at[pl.ds(cid * _NPAD + off, zs)])

    return sk(msg3, dst, zeros).reshape(_NC, _NPAD, d)


def _edge_mlp(xs, ea, w1a, w1b, b1, w2, b2, e, d, h):
    be = 8000
    grid = e // be

    bf = jnp.bfloat16

    def body(xs_ref, ea_ref, w1a_ref, w1b_ref, b1_ref, w2_ref, b2_ref, o_ref):
        cat = jnp.concatenate(
            [xs_ref[...].astype(bf), ea_ref[...].astype(bf)], axis=1)
        w1 = jnp.concatenate([w1a_ref[...], w1b_ref[...]], axis=0)
        hv = jnp.dot(cat, w1, preferred_element_type=jnp.float32) + b1_ref[...]
        hv = jnp.maximum(hv, 0.0)
        o_ref[...] = (jnp.dot(hv.astype(bf), w2_ref[...],
                              preferred_element_type=jnp.float32)
                      + b2_ref[...])

    return pl.pallas_call(
        body,
        grid=(grid,),
        in_specs=[
            pl.BlockSpec((be, d), lambda i: (i, 0)),
            pl.BlockSpec((be, d), lambda i: (i, 0)),
            pl.BlockSpec((d, h), lambda i: (0, 0)),
            pl.BlockSpec((d, h), lambda i: (0, 0)),
            pl.BlockSpec((1, h), lambda i: (0, 0)),
            pl.BlockSpec((h, d), lambda i: (0, 0)),
            pl.BlockSpec((1, d), lambda i: (0, 0)),
        ],
        out_specs=pl.BlockSpec((be, d), lambda i: (i, 0)),
        out_shape=jax.ShapeDtypeStruct((e, d), jnp.float32),
    )(xs, ea, w1a, w1b, b1, w2, b2)


def _node_mlp(x, p0, p1, c0, c1, u, batch3, w1a, w1b, w1c, b1, w2, b2,
              n, d, nb, gd, h):
    bn = 1000
    grid = n // bn

    def body(x_ref, p0_ref, p1_ref, c0_ref, c1_ref, u_ref, b_ref,
             w1a_ref, w1b_ref, w1c_ref, b1_ref, w2_ref, b2_ref,
             y_ref, s0_ref, s1_ref, s2_ref):
        i = pl.program_id(0)
        xv = x_ref[...]
        p = p0_ref[...] + p1_ref[...]
        cnt = jnp.max(c0_ref[...] + c1_ref[...], axis=1, keepdims=True)
        agg = p / jnp.clip(cnt, 1.0, None)
        bv = b_ref[0, 0, :]
        oh = (bv[:, None] == lax.broadcasted_iota(jnp.int32, (bn, nb), 1)
              ).astype(jnp.float32)
        ub = jnp.dot(oh, u_ref[...], preferred_element_type=jnp.float32)
        hv = (jnp.dot(xv, w1a_ref[...], preferred_element_type=jnp.float32)
              + jnp.dot(agg, w1b_ref[...], preferred_element_type=jnp.float32)
              + jnp.dot(ub, w1c_ref[...], preferred_element_type=jnp.float32)
              + b1_ref[...])
        hv = jnp.maximum(hv, 0.0)
        y = (jnp.dot(hv, w2_ref[...], preferred_element_type=jnp.float32)
             + b2_ref[...] + xv)
        y_ref[...] = y

        dg = lambda a, b: lax.dot_general(
            a, b, (((0,), (0,)), ((), ())), preferred_element_type=jnp.float32)

        @pl.when(i == 0)
        def _():
            s0_ref[...] = jnp.zeros_like(s0_ref)
            s1_ref[...] = jnp.zeros_like(s1_ref)
            s2_ref[...] = jnp.zeros_like(s2_ref)

        s0_ref[...] += dg(oh, jnp.ones_like(y))
        s1_ref[...] += dg(oh, y)
        s2_ref[...] += dg(oh, y * y)

    return pl.pallas_call(
        body,
        grid=(grid,),
        in_specs=[
            pl.BlockSpec((bn, d), lambda i: (i, 0)),
            pl.BlockSpec((bn, d), lambda i: (i, 0)),
            pl.BlockSpec((bn, d), lambda i: (i, 0)),
            pl.BlockSpec((bn, d), lambda i: (i, 0)),
            pl.BlockSpec((bn, d), lambda i: (i, 0)),
            pl.BlockSpec((nb, gd), lambda i: (0, 0)),
            pl.BlockSpec((1, 1, bn), lambda i: (i, 0, 0)),
            pl.BlockSpec((d, h), lambda i: (0, 0)),
            pl.BlockSpec((d, h), lambda i: (0, 0)),
            pl.BlockSpec((gd, h), lambda i: (0, 0)),
            pl.BlockSpec((1, h), lambda i: (0, 0)),
            pl.BlockSpec((h, d), lambda i: (0, 0)),
            pl.BlockSpec((1, d), lambda i: (0, 0)),
        ],
        out_specs=[
            pl.BlockSpec((bn, d), lambda i: (i, 0)),
            pl.BlockSpec((nb, d), lambda i: (0, 0)),
            pl.BlockSpec((nb, d), lambda i: (0, 0)),
            pl.BlockSpec((nb, d), lambda i: (0, 0)),
        ],
        out_shape=[
            jax.ShapeDtypeStruct((n, d), jnp.float32),
            jax.ShapeDtypeStruct((nb, d), jnp.float32),
            jax.ShapeDtypeStruct((nb, d), jnp.float32),
            jax.ShapeDtypeStruct((nb, d), jnp.float32),
        ],
    )(x, p0, p1, c0, c1, u, batch3, w1a, w1b, w1c, b1, w2, b2)


def _graphnorm(y, batch3, s0, s1, s2, alpha, gamma, beta, n, d, nb):
    bn = 1000
    grid = n // bn

    def body(y_ref, b_ref, s0_ref, s1_ref, s2_ref, al_ref, ga_ref, be_ref,
             o_ref):
        gcnt = jnp.clip(s0_ref[...], 1.0, None)
        mean = s1_ref[...] / gcnt
        am = al_ref[...] * mean
        var = s2_ref[...] / gcnt - am * (2.0 * mean - am)
        scale = ga_ref[...] / jnp.sqrt(var + 1e-5)
        shift = be_ref[...] - scale * am
        bv = b_ref[0, 0, :]
        oh = (bv[:, None] == lax.broadcasted_iota(jnp.int32, (bn, nb), 1)
              ).astype(jnp.float32)
        o_ref[...] = (jnp.dot(oh, scale, preferred_element_type=jnp.float32)
                      * y_ref[...]
                      + jnp.dot(oh, shift, preferred_element_type=jnp.float32))

    return pl.pallas_call(
        body,
        grid=(grid,),
        in_specs=[
            pl.BlockSpec((bn, d), lambda i: (i, 0)),
            pl.BlockSpec((1, 1, bn), lambda i: (i, 0, 0)),
            pl.BlockSpec((nb, d), lambda i: (0, 0)),
            pl.BlockSpec((nb, d), lambda i: (0, 0)),
            pl.BlockSpec((nb, d), lambda i: (0, 0)),
            pl.BlockSpec((1, d), lambda i: (0, 0)),
            pl.BlockSpec((1, d), lambda i: (0, 0)),
            pl.BlockSpec((1, d), lambda i: (0, 0)),
        ],
        out_specs=pl.BlockSpec((bn, d), lambda i: (i, 0)),
        out_shape=jax.ShapeDtypeStruct((n, d), jnp.float32),
    )(y, batch3, s0, s1, s2, alpha, gamma, beta)


def kernel(x, edge_index, edge_attr, u, batch, agg_w1, agg_b1, agg_w2, agg_b2,
           upd_w1, upd_b1, upd_w2, upd_b2, gn_alpha, gn_gamma, gn_beta):
    n, d = x.shape
    e = edge_index.shape[1]
    nb, gd = u.shape
    h = agg_w1.shape[1]
    hu = upd_w1.shape[1]

    src = edge_index[0]
    dst = edge_index[1]

    xs, cnts = _sc_gather_count(x, src, dst, n, e, d)

    bf = jnp.bfloat16
    msg = _edge_mlp(xs, edge_attr, agg_w1[:d].astype(bf), agg_w1[d:].astype(bf),
                    agg_b1.reshape(1, h), agg_w2.astype(bf),
                    agg_b2.reshape(1, d), e, d, h)

    parts = _sc_scatter(msg, dst, n, e, d)

    batch3 = batch.reshape(n // 1000, 1, 1000)
    y, s0, s1, s2 = _node_mlp(
        x, parts[0], parts[1], cnts[0], cnts[1], u, batch3,
        upd_w1[:d], upd_w1[d:2 * d], upd_w1[2 * d:],
        upd_b1.reshape(1, hu), upd_w2, upd_b2.reshape(1, d),
        n, d, nb, gd, hu)

    return _graphnorm(y, batch3, s0, s1, s2,
                      gn_alpha.reshape(1, d), gn_gamma.reshape(1, d),
                      gn_beta.reshape(1, d), n, d, nb)
